# baseline (speedup 1.0000x reference)
import jax
import jax.numpy as jnp
from jax.experimental import pallas as pl
from jax.experimental.pallas import tpu as pltpu


def _head_kernel(pooled_ref, w_ref, b_ref, o_ref):
    o_ref[...] = jnp.dot(pooled_ref[...], w_ref[...],
                         preferred_element_type=jnp.float32) + b_ref[...]


def _gat(h_in, src, dst, W, a_src, a_dst, b, n):
    h = h_in @ W
    alpha_src = (h * a_src).sum(-1)
    alpha_dst = (h * a_dst).sum(-1)
    e = jax.nn.leaky_relu(jnp.take(alpha_src, src) + jnp.take(alpha_dst, dst),
                          negative_slope=0.2)
    m = jax.ops.segment_max(e, dst, num_segments=n)
    m = jnp.where(jnp.isfinite(m), m, 0.0)
    ex = jnp.exp(e - jnp.take(m, dst))
    den = jax.ops.segment_sum(ex, dst, num_segments=n)
    alpha = ex / (jnp.take(den, dst) + 1e-16)
    msg = alpha[:, None] * jnp.take(h, src, axis=0)
    return jax.ops.segment_sum(msg, dst, num_segments=n) + b


def kernel(x, edge_index, batch, atom_emb, W1, a1_src, a1_dst, b1, W2, a2_src,
           a2_dst, b2, W3, a3_src, a3_dst, b3, W_lin, b_lin):
    n = x.shape[0]
    G = 128
    h = jnp.zeros((n, atom_emb.shape[2]), dtype=atom_emb.dtype)
    for f in range(atom_emb.shape[0]):
        h = h + jnp.take(atom_emb[f], x[:, f], axis=0)
    loop = jnp.arange(n)
    src = jnp.concatenate([edge_index[0], loop])
    dst = jnp.concatenate([edge_index[1], loop])
    h = jax.nn.relu(_gat(h, src, dst, W1, a1_src, a1_dst, b1, n))
    h = jax.nn.relu(_gat(h, src, dst, W2, a2_src, a2_dst, b2, n))
    h = jax.nn.relu(_gat(h, src, dst, W3, a3_src, a3_dst, b3, n))
    counts = jax.ops.segment_sum(jnp.ones((n,), dtype=h.dtype), batch,
                                 num_segments=G)
    x_add = jax.ops.segment_sum(h, batch, num_segments=G)
    x_mean = x_add / jnp.maximum(counts, 1.0)[:, None]
    x_max = jax.ops.segment_max(h, batch, num_segments=G)
    x_max = jnp.where(counts[:, None] > 0, x_max, 0.0)
    pooled = jnp.concatenate([x_mean, x_add, x_max], axis=1)
    return pl.pallas_call(
        _head_kernel,
        out_shape=jax.ShapeDtypeStruct((G, W_lin.shape[1]), jnp.float32),
    )(pooled, W_lin, b_lin)


# speedup vs baseline: 23.3478x; 23.3478x over previous
import jax
import jax.numpy as jnp
from jax import lax
from jax.experimental import pallas as pl
from jax.experimental.pallas import tpu as pltpu
from jax.experimental.pallas import tpu_sc as plsc

_N = 50000
_E = 800000
_G = 128
_EB = 128
_NBLK = 391
_NE_T = _EB * _NBLK
_EPAD = 16 * _NE_T
_PN = 51200
_STRIPE = _PN // 16
_PH0 = 50176
_BN = 1000
_GRID = _N // _BN
_F32 = jnp.float32
_I32 = jnp.int32
_NEG = -3.4e38


def _emb_body(xf, tabf, offs, h0, ov, xv, idxv, rows, hout, sem1, sem2):
    c = lax.axis_index("c")
    s = lax.axis_index("s")
    wid = c * 16 + s
    pltpu.sync_copy(offs, ov)

    def blk(nb, _):
        node0 = wid * 1568 + nb * 16
        off = node0 * 9
        pltpu.sync_copy(xf.at[pl.ds(off, 144)], xv)
        for q in range(9):
            idxv[pl.ds(16 * q, 16)] = xv[pl.ds(16 * q, 16)] + ov[pl.ds(16 * q, 16)]
        cp1 = pltpu.async_copy(tabf.at[idxv.at[pl.ds(0, 128)]],
                               rows.at[pl.ds(0, 128)], sem1)
        cp2 = pltpu.async_copy(tabf.at[idxv.at[pl.ds(128, 16)]],
                               rows.at[pl.ds(128, 16)], sem2)
        cp1.wait()
        cp2.wait()

        def nrow(r, _2):
            rr = r * 9
            for q in range(8):
                acc = rows[rr, pl.ds(16 * q, 16)]
                for t in range(1, 9):
                    acc = acc + rows[rr + t, pl.ds(16 * q, 16)]
                hout[r, pl.ds(16 * q, 16)] = acc
            return 0

        lax.fori_loop(0, 16, nrow, 0)
        pltpu.sync_copy(hout, h0.at[pl.ds(node0, 16)])
        return 0

    lax.fori_loop(0, 98, blk, 0)


def _edge_body(hwsf, asf, adf, mv, srcP, dstP, numP, denP,
               accS, denS, z32, zden, srcv, dstv, gidx, asv, adv, exv, rows,
               mvv, sem1, sem2, sem3):
    c = lax.axis_index("c")
    s = lax.axis_index("s")
    zf = jnp.zeros((16,), _F32)

    def z1(i, _):
        z32[i, pl.ds(0, 16)] = zf
        z32[i, pl.ds(16, 16)] = zf
        return 0

    lax.fori_loop(0, _EB, z1, 0)

    def z2(i, _):
        zden[pl.ds(i * 16, 16)] = zf
        return 0

    lax.fori_loop(0, _STRIPE // 16, z2, 0)
    r0 = s * _STRIPE
    for k in range(_STRIPE // _EB):
        pltpu.sync_copy(z32, accS.at[pl.ds(r0 + k * _EB, _EB)])
    pltpu.sync_copy(zden, denS.at[pl.ds(r0, _STRIPE)])
    pltpu.sync_copy(mv, mvv)
    plsc.subcore_barrier()

    coff = c * _N
    e0 = s * _NE_T

    def blk(b, _):
        base = e0 + b * _EB
        pltpu.sync_copy(srcP.at[pl.ds(base, _EB)], srcv)
        pltpu.sync_copy(dstP.at[pl.ds(base, _EB)], dstv)
        for q in range(8):
            gidx[pl.ds(16 * q, 16)] = srcv[pl.ds(16 * q, 16)] + coff
        cp1 = pltpu.async_copy(asf.at[srcv], asv, sem1)
        cp2 = pltpu.async_copy(adf.at[dstv], adv, sem2)
        cp3 = pltpu.async_copy(hwsf.at[gidx], rows, sem3)
        cp1.wait()
        cp2.wait()
        mvec = mvv[pl.ds(0, 16)]
        for q in range(8):
            t = asv[pl.ds(16 * q, 16)] + adv[pl.ds(16 * q, 16)]
            e = jnp.where(t >= 0.0, t, 0.2 * t)
            exv[pl.ds(16 * q, 16)] = jnp.exp(e - mvec)
        cp3.wait()

        def scg(g, _2):
            exvec = exv[pl.ds(g * 16, 16)]
            for r in range(16):
                sp = lax.gather(
                    exvec, jnp.full((16, 1), r, _I32),
                    lax.GatherDimensionNumbers(
                        offset_dims=(), collapsed_slice_dims=(0,),
                        start_index_map=(0,)),
                    slice_sizes=(1,),
                    mode=lax.GatherScatterMode.PROMISE_IN_BOUNDS)
                rr = g * 16 + r
                rows[rr, pl.ds(0, 16)] = rows[rr, pl.ds(0, 16)] * sp
                rows[rr, pl.ds(16, 16)] = rows[rr, pl.ds(16, 16)] * sp
            return 0

        lax.fori_loop(0, _EB // 16, scg, 0)
        pltpu.sync_copy(rows, accS.at[dstv], add=True)
        pltpu.sync_copy(exv, denS.at[dstv], add=True)
        return 0

    lax.fori_loop(0, _NBLK, blk, 0)
    plsc.subcore_barrier()
    pltpu.sync_copy(accS.at[pl.ds(r0, _STRIPE)], numP.at[c, pl.ds(r0, _STRIPE)])
    pltpu.sync_copy(denS.at[pl.ds(r0, _STRIPE)], denP.at[c, pl.ds(r0, _STRIPE)])


def _pool_body(h3, bnd, psum, pmax, bndv, rbuf, rowb, osum, omax, sem):
    c = lax.axis_index("c")
    s = lax.axis_index("s")
    wid = c * 16 + s
    pltpu.sync_copy(bnd, bndv)
    iota = lax.iota(_I32, 16)

    def bval(t):
        blkoff = (t // 16) * 16
        vec = bndv[pl.ds(blkoff, 16)]
        return jnp.sum(jnp.where(iota == (t - blkoff), vec, 0))

    zf = jnp.zeros((16,), _F32)
    neg = jnp.full((16,), _NEG, _F32)
    for j in range(4):
        g = wid * 4 + j
        st = bval(g)
        en = bval(g + 1)
        cnt = en - st
        nfull = cnt // 16

        def fb(i, car):
            pltpu.sync_copy(h3.at[pl.ds(st + i * 16, 16)], rbuf)
            ss = list(car[:4])
            mm = list(car[4:])
            for r in range(16):
                for q in range(4):
                    v = rbuf[r, pl.ds(16 * q, 16)]
                    ss[q] = ss[q] + v
                    mm[q] = jnp.maximum(mm[q], v)
            return (*ss, *mm)

        car = lax.fori_loop(0, nfull, fb, (zf, zf, zf, zf, neg, neg, neg, neg))

        def tb(i, car2):
            pltpu.sync_copy(h3.at[pl.ds(st + nfull * 16 + i, 1)], rowb)
            ss = list(car2[:4])
            mm = list(car2[4:])
            for q in range(4):
                v = rowb[0, pl.ds(16 * q, 16)]
                ss[q] = ss[q] + v
                mm[q] = jnp.maximum(mm[q], v)
            return (*ss, *mm)

        car = lax.fori_loop(0, cnt - nfull * 16, tb, car)
        for q in range(4):
            osum[0, pl.ds(16 * q, 16)] = car[q]
            omax[0, pl.ds(16 * q, 16)] = car[4 + q]
        pltpu.sync_copy(osum, psum.at[pl.ds(g, 1)])
        pltpu.sync_copy(omax, pmax.at[pl.ds(g, 1)])


def _prep_tail(i, hout, wn_ref, ans_ref, adn_ref,
               hwn_ref, asn_ref, adn_out, mxs_ref, mxd_ref):
    hwn = jnp.dot(hout, wn_ref[...], preferred_element_type=_F32)
    hwn_ref[...] = hwn
    a_s = jnp.sum(hwn * ans_ref[...], axis=1)
    a_d = jnp.sum(hwn * adn_ref[...], axis=1)
    asn_ref[...] = a_s.reshape(1, 1, _BN)
    adn_out[...] = a_d.reshape(1, 1, _BN)
    ms_prev = jnp.where(i == 0, _NEG, mxs_ref[...][0, 0])
    md_prev = jnp.where(i == 0, _NEG, mxd_ref[...][0, 0])
    mxs_ref[...] = jnp.maximum(ms_prev, jnp.max(a_s)).reshape(1, 1)
    mxd_ref[...] = jnp.maximum(md_prev, jnp.max(a_d)).reshape(1, 1)


def _prep1_body(h0_ref, batch_ref, w_ref, ans_ref, adn_ref,
                hwn_ref, asn_ref, adn_out, mxs_ref, mxd_ref, cnt_ref):
    i = pl.program_id(0)
    _prep_tail(i, h0_ref[...], w_ref, ans_ref, adn_ref,
               hwn_ref, asn_ref, adn_out, mxs_ref, mxd_ref)
    b = batch_ref[...].reshape(_BN)
    oh = (b[:, None] == lax.broadcasted_iota(_I32, (_BN, _G), 1)).astype(_F32)
    c_prev = jnp.where(i == 0, jnp.zeros((1, _G), _F32), cnt_ref[...])
    cnt_ref[...] = c_prev + jnp.sum(oh, axis=0, keepdims=True)


def _combine(num0_ref, num1_ref, den_ref, hw_ref, as_ref, ad_ref,
             mxs_ref, mxd_ref, b_ref):
    mx = mxs_ref[...][0, 0] + mxd_ref[...][0, 0]
    m = jnp.maximum(mx, 0.2 * mx)
    t = as_ref[...].reshape(_BN) + ad_ref[...].reshape(_BN)
    e = jnp.where(t >= 0.0, t, 0.2 * t)
    exs = jnp.exp(e - m)
    hw = hw_ref[...]
    num = jnp.concatenate([num0_ref[0], num1_ref[0]], axis=1)
    numv = num + exs[:, None] * hw
    denv = den_ref[...].reshape(_BN) + exs + 1e-16
    return jnp.maximum(numv / denv[:, None] + b_ref[...], 0.0)


def _cp_body(num0_ref, num1_ref, den_ref, hw_ref, as_ref, ad_ref,
             mxs_ref, mxd_ref, b_ref, wn_ref, ans_ref, adn_ref,
             hwn_ref, asn_ref, adn_out, mxs_out, mxd_out):
    i = pl.program_id(0)
    hout = _combine(num0_ref, num1_ref, den_ref, hw_ref, as_ref, ad_ref,
                    mxs_ref, mxd_ref, b_ref)
    _prep_tail(i, hout, wn_ref, ans_ref, adn_ref,
               hwn_ref, asn_ref, adn_out, mxs_out, mxd_out)


def _comb3_body(num0_ref, num1_ref, den_ref, hw_ref, as_ref, ad_ref,
                mxs_ref, mxd_ref, b_ref, cnt_ref, h3_ref, bnd_ref):
    h3_ref[...] = _combine(num0_ref, num1_ref, den_ref, hw_ref, as_ref,
                           ad_ref, mxs_ref, mxd_ref, b_ref)
    cnts = cnt_ref[...]
    ii = lax.broadcasted_iota(_I32, (_G, _G), 0)
    jj = lax.broadcasted_iota(_I32, (_G, _G), 1)
    tri = (ii <= jj).astype(_F32)
    cs = jnp.dot(cnts, tri, preferred_element_type=_F32)
    excl = (cs - cnts).astype(_I32)
    bnd_ref[...] = jnp.concatenate(
        [excl, jnp.full((1, 16), _N, _I32)], axis=1)


def _head_body(psum_ref, pmax_ref, cnt_ref, wl_ref, bl_ref, o_ref):
    cc = cnt_ref[...].reshape(_G, 1)
    ps = psum_ref[...]
    mean = ps / jnp.maximum(cc, 1.0)
    mx = jnp.where(cc > 0, pmax_ref[...], 0.0)
    pooled = jnp.concatenate([mean, ps, mx], axis=1)
    o_ref[...] = jnp.dot(pooled, wl_ref[...],
                         preferred_element_type=_F32) + bl_ref[...]


def _sc_mesh():
    return plsc.VectorSubcoreMesh(core_axis_name="c", subcore_axis_name="s")


def _emb_call(xf, tabf, offs):
    kfn = pl.kernel(
        _emb_body,
        out_type=jax.ShapeDtypeStruct((_PH0, 128), _F32),
        mesh=_sc_mesh(),
        scratch_types=[
            pltpu.VMEM((144,), _I32),
            pltpu.VMEM((144,), _I32),
            pltpu.VMEM((144,), _I32),
            pltpu.VMEM((144, 128), _F32),
            pltpu.VMEM((16, 128), _F32),
            pltpu.SemaphoreType.DMA,
            pltpu.SemaphoreType.DMA,
        ],
    )
    return kfn(xf, tabf, offs)


def _edge_call(hW, as3, ad3, mxs, mxd, srcP, dstP):
    asf = as3.reshape(_N)
    adf = ad3.reshape(_N)
    hwsf = jnp.concatenate([hW[:, :32], hW[:, 32:]], axis=0)
    mx = mxs[0, 0] + mxd[0, 0]
    m = jnp.maximum(mx, 0.2 * mx)
    mv = jnp.full((16,), 1.0, _F32) * m
    kfn = pl.kernel(
        _edge_body,
        out_type=[jax.ShapeDtypeStruct((2, _PN, 32), _F32),
                  jax.ShapeDtypeStruct((2, _PN), _F32)],
        mesh=_sc_mesh(),
        scratch_types=[
            pltpu.VMEM_SHARED((_PN, 32), _F32),
            pltpu.VMEM_SHARED((_PN,), _F32),
            pltpu.VMEM((_EB, 32), _F32),
            pltpu.VMEM((_STRIPE,), _F32),
            pltpu.VMEM((_EB,), _I32),
            pltpu.VMEM((_EB,), _I32),
            pltpu.VMEM((_EB,), _I32),
            pltpu.VMEM((_EB,), _F32),
            pltpu.VMEM((_EB,), _F32),
            pltpu.VMEM((_EB,), _F32),
            pltpu.VMEM((_EB, 32), _F32),
            pltpu.VMEM((16,), _F32),
            pltpu.SemaphoreType.DMA,
            pltpu.SemaphoreType.DMA,
            pltpu.SemaphoreType.DMA,
        ],
        compiler_params=pltpu.CompilerParams(use_tc_tiling_on_sc=False),
    )
    return kfn(hwsf, asf, adf, mv, srcP, dstP)


def _pool_call(h3, bnd):
    kfn = pl.kernel(
        _pool_body,
        out_type=[jax.ShapeDtypeStruct((_G, 64), _F32),
                  jax.ShapeDtypeStruct((_G, 64), _F32)],
        mesh=_sc_mesh(),
        scratch_types=[
            pltpu.VMEM((144,), _I32),
            pltpu.VMEM((16, 64), _F32),
            pltpu.VMEM((1, 64), _F32),
            pltpu.VMEM((1, 64), _F32),
            pltpu.VMEM((1, 64), _F32),
            pltpu.SemaphoreType.DMA,
        ],
        compiler_params=pltpu.CompilerParams(use_tc_tiling_on_sc=False,
                                             needs_layout_passes=False),
    )
    return kfn(h3, bnd)


def _full(shape):
    return pl.BlockSpec(shape, lambda b: tuple(0 for _ in shape))


def _prep1_call(h0, batch3, w1, a1s, a1d):
    return pl.pallas_call(
        _prep1_body,
        grid=(_GRID,),
        in_specs=[
            pl.BlockSpec((_BN, 128), lambda b: (b, 0)),
            pl.BlockSpec((1, 1, _BN), lambda b: (b, 0, 0)),
            _full((128, 64)),
            _full((1, 64)),
            _full((1, 64)),
        ],
        out_specs=[
            pl.BlockSpec((_BN, 64), lambda b: (b, 0)),
            pl.BlockSpec((1, 1, _BN), lambda b: (b, 0, 0)),
            pl.BlockSpec((1, 1, _BN), lambda b: (b, 0, 0)),
            _full((1, 1)),
            _full((1, 1)),
            _full((1, _G)),
        ],
        out_shape=[
            jax.ShapeDtypeStruct((_N, 64), _F32),
            jax.ShapeDtypeStruct((_GRID, 1, _BN), _F32),
            jax.ShapeDtypeStruct((_GRID, 1, _BN), _F32),
            jax.ShapeDtypeStruct((1, 1), _F32),
            jax.ShapeDtypeStruct((1, 1), _F32),
            jax.ShapeDtypeStruct((1, _G), _F32),
        ],
    )(h0, batch3, w1, a1s, a1d)


_CP_IN_SPECS = [
    pl.BlockSpec((1, _BN, 32), lambda b: (0, b, 0)),
    pl.BlockSpec((1, _BN, 32), lambda b: (1, b, 0)),
    pl.BlockSpec((1, 1, _BN), lambda b: (b, 0, 0)),
    pl.BlockSpec((_BN, 64), lambda b: (b, 0)),
    pl.BlockSpec((1, 1, _BN), lambda b: (b, 0, 0)),
    pl.BlockSpec((1, 1, _BN), lambda b: (b, 0, 0)),
    _full((1, 1)),
    _full((1, 1)),
    _full((1, 64)),
]


def _cp_call(numP, denP, hW, as3, ad3, mxs, mxd, bb, wn, ans, adn):
    den3 = denP[0, :_N].reshape(_GRID, 1, _BN)
    return pl.pallas_call(
        _cp_body,
        grid=(_GRID,),
        in_specs=_CP_IN_SPECS + [
            _full((64, 64)),
            _full((1, 64)),
            _full((1, 64)),
        ],
        out_specs=[
            pl.BlockSpec((_BN, 64), lambda b: (b, 0)),
            pl.BlockSpec((1, 1, _BN), lambda b: (b, 0, 0)),
            pl.BlockSpec((1, 1, _BN), lambda b: (b, 0, 0)),
            _full((1, 1)),
            _full((1, 1)),
        ],
        out_shape=[
            jax.ShapeDtypeStruct((_N, 64), _F32),
            jax.ShapeDtypeStruct((_GRID, 1, _BN), _F32),
            jax.ShapeDtypeStruct((_GRID, 1, _BN), _F32),
            jax.ShapeDtypeStruct((1, 1), _F32),
            jax.ShapeDtypeStruct((1, 1), _F32),
        ],
    )(numP, numP, den3, hW, as3, ad3, mxs, mxd, bb.reshape(1, 64),
      wn, ans.reshape(1, 64), adn.reshape(1, 64))


def _comb3_call(numP, denP, hW, as3, ad3, mxs, mxd, b3, counts):
    den3 = denP[0, :_N].reshape(_GRID, 1, _BN)
    return pl.pallas_call(
        _comb3_body,
        grid=(_GRID,),
        in_specs=_CP_IN_SPECS + [_full((1, _G))],
        out_specs=[
            pl.BlockSpec((_BN, 64), lambda b: (b, 0)),
            _full((1, 144)),
        ],
        out_shape=[
            jax.ShapeDtypeStruct((_N, 64), _F32),
            jax.ShapeDtypeStruct((1, 144), _I32),
        ],
    )(numP, numP, den3, hW, as3, ad3, mxs, mxd, b3.reshape(1, 64), counts)


def _head_call(psum, pmax, counts, w_lin, b_lin):
    return pl.pallas_call(
        _head_body,
        out_shape=jax.ShapeDtypeStruct((_G, w_lin.shape[1]), _F32),
    )(psum, pmax, counts, w_lin, b_lin.reshape(1, -1))


def kernel(x, edge_index, batch, atom_emb, W1, a1_src, a1_dst, b1, W2, a2_src,
           a2_dst, b2, W3, a3_src, a3_dst, b3, W_lin, b_lin):
    x = x.astype(_I32)
    xf = jnp.pad(x.reshape(-1), (0, _PH0 * 9 - _N * 9))
    tabf = atom_emb.reshape(900, 128)
    offs = jnp.tile(jnp.arange(9, dtype=_I32) * 100, 16)
    h0full = _emb_call(xf, tabf, offs)

    srcP = jnp.concatenate(
        [edge_index[0].astype(_I32), jnp.zeros((_EPAD - _E,), _I32)])
    dstP = jnp.concatenate(
        [edge_index[1].astype(_I32), jnp.full((_EPAD - _E,), _N, _I32)])
    batch3 = batch.astype(_I32).reshape(_GRID, 1, _BN)

    hW, as3, ad3, mxs, mxd, counts = _prep1_call(
        h0full[:_N], batch3, W1, a1_src.reshape(1, 64), a1_dst.reshape(1, 64))

    for (bb, wn, ans, adn) in ((b1, W2, a2_src, a2_dst),
                               (b2, W3, a3_src, a3_dst)):
        numP, denP = _edge_call(hW, as3, ad3, mxs, mxd, srcP, dstP)
        hW, as3, ad3, mxs, mxd = _cp_call(
            numP, denP, hW, as3, ad3, mxs, mxd, bb, wn, ans, adn)

    numP, denP = _edge_call(hW, as3, ad3, mxs, mxd, srcP, dstP)
    h3, bnd = _comb3_call(numP, denP, hW, as3, ad3, mxs, mxd, b3, counts)

    psum, pmax = _pool_call(h3, bnd.reshape(144))
    return _head_call(psum, pmax, counts, W_lin, b_lin)


# speedup vs baseline: 40.8688x; 1.7504x over previous
import jax
import jax.numpy as jnp
from jax import lax
from jax.experimental import pallas as pl
from jax.experimental.pallas import tpu as pltpu
from jax.experimental.pallas import tpu_sc as plsc

_N = 50000
_E = 800000
_G = 128
_EB = 128
_NBLK = 392
_NE_T = _EB * _NBLK
_EPAD = 16 * _NE_T
_PN = 51200
_STRIPE = _PN // 16
_PH0 = 50176
_BN = 1000
_GRID = _N // _BN
_F32 = jnp.float32
_I32 = jnp.int32
_NEG = -3.4e38


def _emb_body(xf, tabf, offs, h0, ov, xv, idxv, rows, hout, sem1, sem2):
    c = lax.axis_index("c")
    s = lax.axis_index("s")
    wid = c * 16 + s
    pltpu.sync_copy(offs, ov)

    def blk(nb, _):
        node0 = wid * 1568 + nb * 16
        off = node0 * 9
        pltpu.sync_copy(xf.at[pl.ds(off, 144)], xv)
        for q in range(9):
            idxv[pl.ds(16 * q, 16)] = xv[pl.ds(16 * q, 16)] + ov[pl.ds(16 * q, 16)]
        cp1 = pltpu.async_copy(tabf.at[idxv.at[pl.ds(0, 128)]],
                               rows.at[pl.ds(0, 128)], sem1)
        cp2 = pltpu.async_copy(tabf.at[idxv.at[pl.ds(128, 16)]],
                               rows.at[pl.ds(128, 16)], sem2)
        cp1.wait()
        cp2.wait()

        def nrow(r, _2):
            rr = r * 9
            for q in range(8):
                acc = rows[rr, pl.ds(16 * q, 16)]
                for t in range(1, 9):
                    acc = acc + rows[rr + t, pl.ds(16 * q, 16)]
                hout[r, pl.ds(16 * q, 16)] = acc
            return 0

        lax.fori_loop(0, 16, nrow, 0)
        pltpu.sync_copy(hout, h0.at[pl.ds(node0, 16)])
        return 0

    lax.fori_loop(0, 98, blk, 0)


def _edge_body(hwsf, asf, adf, mv, srcP, dstP, numP, denP,
               accS, denS, z32, zden, mvv,
               srcA, dstA, sixA, divA, gixA, asvA, advA, exvA, rowsA,
               srcB, dstB, sixB, divB, gixB, asvB, advB, exvB, rowsB,
               sA1, sA2, sA3, sB1, sB2, sB3, siA1, siA2, siB1, siB2):
    c = lax.axis_index("c")
    s = lax.axis_index("s")
    zf = jnp.zeros((16,), _F32)

    def z1(i, _):
        z32[i, pl.ds(0, 16)] = zf
        z32[i, pl.ds(16, 16)] = zf
        return 0

    lax.fori_loop(0, _EB, z1, 0)

    def z2(i, _):
        zden[pl.ds(i * 16, 16)] = zf
        return 0

    lax.fori_loop(0, _STRIPE // 16, z2, 0)
    r0 = s * _STRIPE
    for k in range(_STRIPE // _EB):
        pltpu.sync_copy(z32, accS.at[pl.ds(r0 + k * _EB, _EB)])
    pltpu.sync_copy(zden, denS.at[pl.ds(r0, _STRIPE)])
    pltpu.sync_copy(mv, mvv)
    plsc.subcore_barrier()

    coff = c * _N
    e0 = s * _NE_T
    nb1 = _NBLK - 1

    def idx_fire(b, sv, dv, s1, s2):
        base = e0 + b * _EB
        pltpu.async_copy(srcP.at[pl.ds(base, _EB)], sv, s1)
        pltpu.async_copy(dstP.at[pl.ds(base, _EB)], dv, s2)

    def idx_wait(sv, dv, s1, s2):
        pltpu.make_async_copy(srcP.at[pl.ds(0, _EB)], sv, s1).wait()
        pltpu.make_async_copy(dstP.at[pl.ds(0, _EB)], dv, s2).wait()

    def gfire(sv, dv, six, div, gix, asv, adv, rows, s1, s2, s3):
        for q in range(8):
            sq = sv[pl.ds(16 * q, 16)]
            six[pl.ds(16 * q, 16)] = sq
            gix[pl.ds(16 * q, 16)] = sq + coff
            div[pl.ds(16 * q, 16)] = dv[pl.ds(16 * q, 16)]
        pltpu.async_copy(asf.at[six], asv, s1)
        pltpu.async_copy(adf.at[div], adv, s2)
        pltpu.async_copy(hwsf.at[gix], rows, s3)

    def gwait(asv, adv, rows, s1, s2, s3):
        pltpu.make_async_copy(asf.at[pl.ds(0, _EB)], asv, s1).wait()
        pltpu.make_async_copy(adf.at[pl.ds(0, _EB)], adv, s2).wait()
        pltpu.make_async_copy(hwsf.at[pl.ds(0, _EB)], rows, s3).wait()

    def consume(div, asv, adv, exv, rows):
        mvec = mvv[pl.ds(0, 16)]
        for q in range(8):
            t = asv[pl.ds(16 * q, 16)] + adv[pl.ds(16 * q, 16)]
            e = jnp.where(t >= 0.0, t, 0.2 * t)
            exv[pl.ds(16 * q, 16)] = jnp.exp(e - mvec)

        def scg(g, _2):
            exvec = exv[pl.ds(g * 16, 16)]
            for r in range(16):
                sp = lax.gather(
                    exvec, jnp.full((16, 1), r, _I32),
                    lax.GatherDimensionNumbers(
                        offset_dims=(), collapsed_slice_dims=(0,),
                        start_index_map=(0,)),
                    slice_sizes=(1,),
                    mode=lax.GatherScatterMode.PROMISE_IN_BOUNDS)
                rr = g * 16 + r
                rows[rr, pl.ds(0, 16)] = rows[rr, pl.ds(0, 16)] * sp
                rows[rr, pl.ds(16, 16)] = rows[rr, pl.ds(16, 16)] * sp
            return 0

        lax.fori_loop(0, _EB // 16, scg, 0)
        pltpu.sync_copy(rows, accS.at[div], add=True)
        pltpu.sync_copy(exv, denS.at[div], add=True)

    pltpu.sync_copy(srcP.at[pl.ds(e0, _EB)], srcA)
    pltpu.sync_copy(dstP.at[pl.ds(e0, _EB)], dstA)
    gfire(srcA, dstA, sixA, divA, gixA, asvA, advA, rowsA, sA1, sA2, sA3)
    idx_fire(1, srcB, dstB, siB1, siB2)

    def it(j, _):
        g0 = 2 * j
        idx_wait(srcB, dstB, siB1, siB2)
        gfire(srcB, dstB, sixB, divB, gixB, asvB, advB, rowsB, sB1, sB2, sB3)
        idx_fire(jnp.minimum(g0 + 2, nb1), srcA, dstA, siA1, siA2)
        gwait(asvA, advA, rowsA, sA1, sA2, sA3)
        consume(divA, asvA, advA, exvA, rowsA)
        idx_wait(srcA, dstA, siA1, siA2)
        gfire(srcA, dstA, sixA, divA, gixA, asvA, advA, rowsA, sA1, sA2, sA3)
        idx_fire(jnp.minimum(g0 + 3, nb1), srcB, dstB, siB1, siB2)
        gwait(asvB, advB, rowsB, sB1, sB2, sB3)
        consume(divB, asvB, advB, exvB, rowsB)
        return 0

    lax.fori_loop(0, _NBLK // 2, it, 0)
    gwait(asvA, advA, rowsA, sA1, sA2, sA3)
    idx_wait(srcB, dstB, siB1, siB2)
    plsc.subcore_barrier()
    pltpu.sync_copy(accS.at[pl.ds(r0, _STRIPE)], numP.at[c, pl.ds(r0, _STRIPE)])
    pltpu.sync_copy(denS.at[pl.ds(r0, _STRIPE)], denP.at[c, pl.ds(r0, _STRIPE)])


def _pool_body(h3, bnd, psum, pmax, bndv, rbuf, rowb, osum, omax, sem):
    c = lax.axis_index("c")
    s = lax.axis_index("s")
    wid = c * 16 + s
    pltpu.sync_copy(bnd, bndv)
    iota = lax.iota(_I32, 16)

    def bval(t):
        blkoff = (t // 16) * 16
        vec = bndv[pl.ds(blkoff, 16)]
        return jnp.sum(jnp.where(iota == (t - blkoff), vec, 0))

    zf = jnp.zeros((16,), _F32)
    neg = jnp.full((16,), _NEG, _F32)
    for j in range(4):
        g = wid * 4 + j
        st = bval(g)
        en = bval(g + 1)
        cnt = en - st
        nfull = cnt // 16

        def fb(i, car):
            pltpu.sync_copy(h3.at[pl.ds(st + i * 16, 16)], rbuf)
            ss = list(car[:4])
            mm = list(car[4:])
            for r in range(16):
                for q in range(4):
                    v = rbuf[r, pl.ds(16 * q, 16)]
                    ss[q] = ss[q] + v
                    mm[q] = jnp.maximum(mm[q], v)
            return (*ss, *mm)

        car = lax.fori_loop(0, nfull, fb, (zf, zf, zf, zf, neg, neg, neg, neg))

        def tb(i, car2):
            pltpu.sync_copy(h3.at[pl.ds(st + nfull * 16 + i, 1)], rowb)
            ss = list(car2[:4])
            mm = list(car2[4:])
            for q in range(4):
                v = rowb[0, pl.ds(16 * q, 16)]
                ss[q] = ss[q] + v
                mm[q] = jnp.maximum(mm[q], v)
            return (*ss, *mm)

        car = lax.fori_loop(0, cnt - nfull * 16, tb, car)
        for q in range(4):
            osum[0, pl.ds(16 * q, 16)] = car[q]
            omax[0, pl.ds(16 * q, 16)] = car[4 + q]
        pltpu.sync_copy(osum, psum.at[pl.ds(g, 1)])
        pltpu.sync_copy(omax, pmax.at[pl.ds(g, 1)])


def _prep_tail(i, hout, wn_ref, ans_ref, adn_ref,
               hwn_ref, asn_ref, adn_out, mxs_ref, mxd_ref):
    hwn = jnp.dot(hout, wn_ref[...], preferred_element_type=_F32)
    hwn_ref[...] = hwn
    a_s = jnp.sum(hwn * ans_ref[...], axis=1)
    a_d = jnp.sum(hwn * adn_ref[...], axis=1)
    asn_ref[...] = a_s.reshape(1, 1, _BN)
    adn_out[...] = a_d.reshape(1, 1, _BN)
    ms_prev = jnp.where(i == 0, _NEG, mxs_ref[...][0, 0])
    md_prev = jnp.where(i == 0, _NEG, mxd_ref[...][0, 0])
    mxs_ref[...] = jnp.maximum(ms_prev, jnp.max(a_s)).reshape(1, 1)
    mxd_ref[...] = jnp.maximum(md_prev, jnp.max(a_d)).reshape(1, 1)


def _prep1_body(h0_ref, batch_ref, w_ref, ans_ref, adn_ref,
                hwn_ref, asn_ref, adn_out, mxs_ref, mxd_ref, cnt_ref):
    i = pl.program_id(0)
    _prep_tail(i, h0_ref[...], w_ref, ans_ref, adn_ref,
               hwn_ref, asn_ref, adn_out, mxs_ref, mxd_ref)
    b = batch_ref[...].reshape(_BN)
    oh = (b[:, None] == lax.broadcasted_iota(_I32, (_BN, _G), 1)).astype(_F32)
    c_prev = jnp.where(i == 0, jnp.zeros((1, _G), _F32), cnt_ref[...])
    cnt_ref[...] = c_prev + jnp.sum(oh, axis=0, keepdims=True)


def _combine(num0_ref, num1_ref, den_ref, hw_ref, as_ref, ad_ref,
             mxs_ref, mxd_ref, b_ref):
    mx = mxs_ref[...][0, 0] + mxd_ref[...][0, 0]
    m = jnp.maximum(mx, 0.2 * mx)
    t = as_ref[...].reshape(_BN) + ad_ref[...].reshape(_BN)
    e = jnp.where(t >= 0.0, t, 0.2 * t)
    exs = jnp.exp(e - m)
    hw = hw_ref[...]
    num = jnp.concatenate([num0_ref[0], num1_ref[0]], axis=1)
    numv = num + exs[:, None] * hw
    denv = den_ref[...].reshape(_BN) + exs + 1e-16
    return jnp.maximum(numv / denv[:, None] + b_ref[...], 0.0)


def _cp_body(num0_ref, num1_ref, den_ref, hw_ref, as_ref, ad_ref,
             mxs_ref, mxd_ref, b_ref, wn_ref, ans_ref, adn_ref,
             hwn_ref, asn_ref, adn_out, mxs_out, mxd_out):
    i = pl.program_id(0)
    hout = _combine(num0_ref, num1_ref, den_ref, hw_ref, as_ref, ad_ref,
                    mxs_ref, mxd_ref, b_ref)
    _prep_tail(i, hout, wn_ref, ans_ref, adn_ref,
               hwn_ref, asn_ref, adn_out, mxs_out, mxd_out)


def _comb3_body(num0_ref, num1_ref, den_ref, hw_ref, as_ref, ad_ref,
                mxs_ref, mxd_ref, b_ref, cnt_ref, h3_ref, bnd_ref):
    h3_ref[...] = _combine(num0_ref, num1_ref, den_ref, hw_ref, as_ref,
                           ad_ref, mxs_ref, mxd_ref, b_ref)
    cnts = cnt_ref[...]
    ii = lax.broadcasted_iota(_I32, (_G, _G), 0)
    jj = lax.broadcasted_iota(_I32, (_G, _G), 1)
    tri = (ii <= jj).astype(_F32)
    cs = jnp.dot(cnts, tri, preferred_element_type=_F32)
    excl = (cs - cnts).astype(_I32)
    bnd_ref[...] = jnp.concatenate(
        [excl, jnp.full((1, 16), _N, _I32)], axis=1)


def _head_body(psum_ref, pmax_ref, cnt_ref, wl_ref, bl_ref, o_ref):
    cc = cnt_ref[...].reshape(_G, 1)
    ps = psum_ref[...]
    mean = ps / jnp.maximum(cc, 1.0)
    mx = jnp.where(cc > 0, pmax_ref[...], 0.0)
    pooled = jnp.concatenate([mean, ps, mx], axis=1)
    o_ref[...] = jnp.dot(pooled, wl_ref[...],
                         preferred_element_type=_F32) + bl_ref[...]


def _sc_mesh():
    return plsc.VectorSubcoreMesh(core_axis_name="c", subcore_axis_name="s")


def _emb_call(xf, tabf, offs):
    kfn = pl.kernel(
        _emb_body,
        out_type=jax.ShapeDtypeStruct((_PH0, 128), _F32),
        mesh=_sc_mesh(),
        scratch_types=[
            pltpu.VMEM((144,), _I32),
            pltpu.VMEM((144,), _I32),
            pltpu.VMEM((144,), _I32),
            pltpu.VMEM((144, 128), _F32),
            pltpu.VMEM((16, 128), _F32),
            pltpu.SemaphoreType.DMA,
            pltpu.SemaphoreType.DMA,
        ],
    )
    return kfn(xf, tabf, offs)


def _edge_call(hW, as3, ad3, mxs, mxd, srcP, dstP):
    asf = as3.reshape(_N)
    adf = ad3.reshape(_N)
    hwsf = jnp.concatenate([hW[:, :32], hW[:, 32:]], axis=0)
    mx = mxs[0, 0] + mxd[0, 0]
    m = jnp.maximum(mx, 0.2 * mx)
    mv = jnp.full((16,), 1.0, _F32) * m
    kfn = pl.kernel(
        _edge_body,
        out_type=[jax.ShapeDtypeStruct((2, _PN, 32), _F32),
                  jax.ShapeDtypeStruct((2, _PN), _F32)],
        mesh=_sc_mesh(),
        scratch_types=(
            [pltpu.VMEM_SHARED((_PN, 32), _F32),
             pltpu.VMEM_SHARED((_PN,), _F32),
             pltpu.VMEM((_EB, 32), _F32),
             pltpu.VMEM((_STRIPE,), _F32),
             pltpu.VMEM((16,), _F32)]
            + 2 * ([pltpu.VMEM((_EB,), _I32)] * 5
                   + [pltpu.VMEM((_EB,), _F32)] * 3
                   + [pltpu.VMEM((_EB, 32), _F32)])
            + [pltpu.SemaphoreType.DMA] * 10
        ),
        compiler_params=pltpu.CompilerParams(use_tc_tiling_on_sc=False),
    )
    return kfn(hwsf, asf, adf, mv, srcP, dstP)


def _pool_call(h3, bnd):
    kfn = pl.kernel(
        _pool_body,
        out_type=[jax.ShapeDtypeStruct((_G, 64), _F32),
                  jax.ShapeDtypeStruct((_G, 64), _F32)],
        mesh=_sc_mesh(),
        scratch_types=[
            pltpu.VMEM((144,), _I32),
            pltpu.VMEM((16, 64), _F32),
            pltpu.VMEM((1, 64), _F32),
            pltpu.VMEM((1, 64), _F32),
            pltpu.VMEM((1, 64), _F32),
            pltpu.SemaphoreType.DMA,
        ],
        compiler_params=pltpu.CompilerParams(use_tc_tiling_on_sc=False,
                                             needs_layout_passes=False),
    )
    return kfn(h3, bnd)


def _full(shape):
    return pl.BlockSpec(shape, lambda b: tuple(0 for _ in shape))


def _prep1_call(h0, batch3, w1, a1s, a1d):
    return pl.pallas_call(
        _prep1_body,
        grid=(_GRID,),
        in_specs=[
            pl.BlockSpec((_BN, 128), lambda b: (b, 0)),
            pl.BlockSpec((1, 1, _BN), lambda b: (b, 0, 0)),
            _full((128, 64)),
            _full((1, 64)),
            _full((1, 64)),
        ],
        out_specs=[
            pl.BlockSpec((_BN, 64), lambda b: (b, 0)),
            pl.BlockSpec((1, 1, _BN), lambda b: (b, 0, 0)),
            pl.BlockSpec((1, 1, _BN), lambda b: (b, 0, 0)),
            _full((1, 1)),
            _full((1, 1)),
            _full((1, _G)),
        ],
        out_shape=[
            jax.ShapeDtypeStruct((_N, 64), _F32),
            jax.ShapeDtypeStruct((_GRID, 1, _BN), _F32),
            jax.ShapeDtypeStruct((_GRID, 1, _BN), _F32),
            jax.ShapeDtypeStruct((1, 1), _F32),
            jax.ShapeDtypeStruct((1, 1), _F32),
            jax.ShapeDtypeStruct((1, _G), _F32),
        ],
    )(h0, batch3, w1, a1s, a1d)


_CP_IN_SPECS = [
    pl.BlockSpec((1, _BN, 32), lambda b: (0, b, 0)),
    pl.BlockSpec((1, _BN, 32), lambda b: (1, b, 0)),
    pl.BlockSpec((1, 1, _BN), lambda b: (b, 0, 0)),
    pl.BlockSpec((_BN, 64), lambda b: (b, 0)),
    pl.BlockSpec((1, 1, _BN), lambda b: (b, 0, 0)),
    pl.BlockSpec((1, 1, _BN), lambda b: (b, 0, 0)),
    _full((1, 1)),
    _full((1, 1)),
    _full((1, 64)),
]


def _cp_call(numP, denP, hW, as3, ad3, mxs, mxd, bb, wn, ans, adn):
    den3 = denP[0, :_N].reshape(_GRID, 1, _BN)
    return pl.pallas_call(
        _cp_body,
        grid=(_GRID,),
        in_specs=_CP_IN_SPECS + [
            _full((64, 64)),
            _full((1, 64)),
            _full((1, 64)),
        ],
        out_specs=[
            pl.BlockSpec((_BN, 64), lambda b: (b, 0)),
            pl.BlockSpec((1, 1, _BN), lambda b: (b, 0, 0)),
            pl.BlockSpec((1, 1, _BN), lambda b: (b, 0, 0)),
            _full((1, 1)),
            _full((1, 1)),
        ],
        out_shape=[
            jax.ShapeDtypeStruct((_N, 64), _F32),
            jax.ShapeDtypeStruct((_GRID, 1, _BN), _F32),
            jax.ShapeDtypeStruct((_GRID, 1, _BN), _F32),
            jax.ShapeDtypeStruct((1, 1), _F32),
            jax.ShapeDtypeStruct((1, 1), _F32),
        ],
    )(numP, numP, den3, hW, as3, ad3, mxs, mxd, bb.reshape(1, 64),
      wn, ans.reshape(1, 64), adn.reshape(1, 64))


def _comb3_call(numP, denP, hW, as3, ad3, mxs, mxd, b3, counts):
    den3 = denP[0, :_N].reshape(_GRID, 1, _BN)
    return pl.pallas_call(
        _comb3_body,
        grid=(_GRID,),
        in_specs=_CP_IN_SPECS + [_full((1, _G))],
        out_specs=[
            pl.BlockSpec((_BN, 64), lambda b: (b, 0)),
            _full((1, 144)),
        ],
        out_shape=[
            jax.ShapeDtypeStruct((_N, 64), _F32),
            jax.ShapeDtypeStruct((1, 144), _I32),
        ],
    )(numP, numP, den3, hW, as3, ad3, mxs, mxd, b3.reshape(1, 64), counts)


def _head_call(psum, pmax, counts, w_lin, b_lin):
    return pl.pallas_call(
        _head_body,
        out_shape=jax.ShapeDtypeStruct((_G, w_lin.shape[1]), _F32),
    )(psum, pmax, counts, w_lin, b_lin.reshape(1, -1))


def kernel(x, edge_index, batch, atom_emb, W1, a1_src, a1_dst, b1, W2, a2_src,
           a2_dst, b2, W3, a3_src, a3_dst, b3, W_lin, b_lin):
    x = x.astype(_I32)
    xf = jnp.pad(x.reshape(-1), (0, _PH0 * 9 - _N * 9))
    tabf = atom_emb.reshape(900, 128)
    offs = jnp.tile(jnp.arange(9, dtype=_I32) * 100, 16)
    h0full = _emb_call(xf, tabf, offs)

    srcP = jnp.concatenate(
        [edge_index[0].astype(_I32), jnp.zeros((_EPAD - _E,), _I32)])
    dstP = jnp.concatenate(
        [edge_index[1].astype(_I32), jnp.full((_EPAD - _E,), _N, _I32)])
    batch3 = batch.astype(_I32).reshape(_GRID, 1, _BN)

    hW, as3, ad3, mxs, mxd, counts = _prep1_call(
        h0full[:_N], batch3, W1, a1_src.reshape(1, 64), a1_dst.reshape(1, 64))

    for (bb, wn, ans, adn) in ((b1, W2, a2_src, a2_dst),
                               (b2, W3, a3_src, a3_dst)):
        numP, denP = _edge_call(hW, as3, ad3, mxs, mxd, srcP, dstP)
        hW, as3, ad3, mxs, mxd = _cp_call(
            numP, denP, hW, as3, ad3, mxs, mxd, bb, wn, ans, adn)

    numP, denP = _edge_call(hW, as3, ad3, mxs, mxd, srcP, dstP)
    h3, bnd = _comb3_call(numP, denP, hW, as3, ad3, mxs, mxd, b3, counts)

    psum, pmax = _pool_call(h3, bnd.reshape(144))
    return _head_call(psum, pmax, counts, W_lin, b_lin)


# speedup vs baseline: 47.5410x; 1.1633x over previous
import jax
import jax.numpy as jnp
from jax import lax
from jax.experimental import pallas as pl
from jax.experimental.pallas import tpu as pltpu
from jax.experimental.pallas import tpu_sc as plsc

_N = 50000
_E = 800000
_G = 128
_EB = 128
_NBLK = 392
_NE_T = _EB * _NBLK
_EPAD = 16 * _NE_T
_PN = 51200
_STRIPE = _PN // 16
_PH0 = 50176
_BN = 1000
_GRID = _N // _BN
_F32 = jnp.float32
_I32 = jnp.int32
_NEG = -3.4e38


def _emb_body(xf, taba, offs, hws, asad, ov,
              xvA, idxA, rowsA, xvB, idxB, rowsB, hout,
              sa1, sa2, sa3, sb1, sb2, sb3):
    c = lax.axis_index("c")
    s = lax.axis_index("s")
    wid = c * 16 + s
    pltpu.sync_copy(offs, ov)

    def fire(b, xv, idxv, rows, s1, s2, s3):
        node0 = wid * 1568 + b * 32
        pltpu.sync_copy(xf.at[pl.ds(node0 * 9, 288)], xv)
        for q in range(18):
            idxv[pl.ds(16 * q, 16)] = (xv[pl.ds(16 * q, 16)]
                                       + ov[pl.ds(16 * q, 16)])
        pltpu.async_copy(taba.at[idxv.at[pl.ds(0, 128)]],
                         rows.at[pl.ds(0, 128)], s1)
        pltpu.async_copy(taba.at[idxv.at[pl.ds(128, 128)]],
                         rows.at[pl.ds(128, 128)], s2)
        pltpu.async_copy(taba.at[idxv.at[pl.ds(256, 32)]],
                         rows.at[pl.ds(256, 32)], s3)

    def consume(b, rows, s1, s2, s3):
        pltpu.make_async_copy(taba.at[pl.ds(0, 128)],
                             rows.at[pl.ds(0, 128)], s1).wait()
        pltpu.make_async_copy(taba.at[pl.ds(0, 128)],
                             rows.at[pl.ds(128, 128)], s2).wait()
        pltpu.make_async_copy(taba.at[pl.ds(0, 32)],
                             rows.at[pl.ds(256, 32)], s3).wait()

        def nrow(r, _2):
            rr = r * 9
            for q in range(5):
                acc = rows[rr, pl.ds(16 * q, 16)]
                for t in range(1, 9):
                    acc = acc + rows[rr + t, pl.ds(16 * q, 16)]
                hout[r, pl.ds(16 * q, 16)] = acc
            return 0

        lax.fori_loop(0, 32, nrow, 0)
        node0 = wid * 1568 + b * 32
        pltpu.sync_copy(hout.at[:, pl.ds(0, 32)], hws.at[0, pl.ds(node0, 32)])
        pltpu.sync_copy(hout.at[:, pl.ds(32, 32)], hws.at[1, pl.ds(node0, 32)])
        pltpu.sync_copy(hout.at[:, pl.ds(64, 16)], asad.at[pl.ds(node0, 32)])

    fire(0, xvA, idxA, rowsA, sa1, sa2, sa3)

    def it(j, _):
        fire(2 * j + 1, xvB, idxB, rowsB, sb1, sb2, sb3)
        consume(2 * j, rowsA, sa1, sa2, sa3)
        fire(2 * j + 2, xvA, idxA, rowsA, sa1, sa2, sa3)
        consume(2 * j + 1, rowsB, sb1, sb2, sb3)
        return 0

    lax.fori_loop(0, 24, it, 0)
    consume(48, rowsA, sa1, sa2, sa3)


def _edge_body(hwsf, asf, adf, mv, srcP, dstP, numP, denP,
               accS, denS, z32, zden, mvv,
               srcA, dstA, sixA, divA, gixA, asvA, advA, exvA, rowsA,
               srcB, dstB, sixB, divB, gixB, asvB, advB, exvB, rowsB,
               sA1, sA2, sA3, sB1, sB2, sB3, siA1, siA2, siB1, siB2):
    c = lax.axis_index("c")
    s = lax.axis_index("s")
    zf = jnp.zeros((16,), _F32)

    def z1(i, _):
        z32[i, pl.ds(0, 16)] = zf
        z32[i, pl.ds(16, 16)] = zf
        return 0

    lax.fori_loop(0, _EB, z1, 0)

    def z2(i, _):
        zden[pl.ds(i * 16, 16)] = zf
        return 0

    lax.fori_loop(0, _STRIPE // 16, z2, 0)
    r0 = s * _STRIPE
    for k in range(_STRIPE // _EB):
        pltpu.sync_copy(z32, accS.at[pl.ds(r0 + k * _EB, _EB)])
    pltpu.sync_copy(zden, denS.at[pl.ds(r0, _STRIPE)])
    pltpu.sync_copy(mv, mvv)
    plsc.subcore_barrier()

    coff = c * _PH0
    e0 = s * _NE_T
    nb1 = _NBLK - 1

    def idx_fire(b, sv, dv, s1, s2):
        base = e0 + b * _EB
        pltpu.async_copy(srcP.at[pl.ds(base, _EB)], sv, s1)
        pltpu.async_copy(dstP.at[pl.ds(base, _EB)], dv, s2)

    def idx_wait(sv, dv, s1, s2):
        pltpu.make_async_copy(srcP.at[pl.ds(0, _EB)], sv, s1).wait()
        pltpu.make_async_copy(dstP.at[pl.ds(0, _EB)], dv, s2).wait()

    def gfire(sv, dv, six, div, gix, asv, adv, rows, s1, s2, s3):
        for q in range(8):
            sq = sv[pl.ds(16 * q, 16)]
            six[pl.ds(16 * q, 16)] = sq
            gix[pl.ds(16 * q, 16)] = sq + coff
            div[pl.ds(16 * q, 16)] = dv[pl.ds(16 * q, 16)]
        pltpu.async_copy(asf.at[six], asv, s1)
        pltpu.async_copy(adf.at[div], adv, s2)
        pltpu.async_copy(hwsf.at[gix], rows, s3)

    def gwait(asv, adv, rows, s1, s2, s3):
        pltpu.make_async_copy(asf.at[pl.ds(0, _EB)], asv, s1).wait()
        pltpu.make_async_copy(adf.at[pl.ds(0, _EB)], adv, s2).wait()
        pltpu.make_async_copy(hwsf.at[pl.ds(0, _EB)], rows, s3).wait()

    def consume(div, asv, adv, exv, rows):
        mvec = mvv[pl.ds(0, 16)]
        for q in range(8):
            t = asv[pl.ds(16 * q, 16)] + adv[pl.ds(16 * q, 16)]
            e = jnp.where(t >= 0.0, t, 0.2 * t)
            exv[pl.ds(16 * q, 16)] = jnp.exp(e - mvec)

        def scg(g, _2):
            exvec = exv[pl.ds(g * 16, 16)]
            for r in range(16):
                sp = lax.gather(
                    exvec, jnp.full((16, 1), r, _I32),
                    lax.GatherDimensionNumbers(
                        offset_dims=(), collapsed_slice_dims=(0,),
                        start_index_map=(0,)),
                    slice_sizes=(1,),
                    mode=lax.GatherScatterMode.PROMISE_IN_BOUNDS)
                rr = g * 16 + r
                rows[rr, pl.ds(0, 16)] = rows[rr, pl.ds(0, 16)] * sp
                rows[rr, pl.ds(16, 16)] = rows[rr, pl.ds(16, 16)] * sp
            return 0

        lax.fori_loop(0, _EB // 16, scg, 0)
        pltpu.sync_copy(rows, accS.at[div], add=True)
        pltpu.sync_copy(exv, denS.at[div], add=True)

    pltpu.sync_copy(srcP.at[pl.ds(e0, _EB)], srcA)
    pltpu.sync_copy(dstP.at[pl.ds(e0, _EB)], dstA)
    gfire(srcA, dstA, sixA, divA, gixA, asvA, advA, rowsA, sA1, sA2, sA3)
    idx_fire(1, srcB, dstB, siB1, siB2)

    def it(j, _):
        g0 = 2 * j
        idx_wait(srcB, dstB, siB1, siB2)
        gfire(srcB, dstB, sixB, divB, gixB, asvB, advB, rowsB, sB1, sB2, sB3)
        idx_fire(jnp.minimum(g0 + 2, nb1), srcA, dstA, siA1, siA2)
        gwait(asvA, advA, rowsA, sA1, sA2, sA3)
        consume(divA, asvA, advA, exvA, rowsA)
        idx_wait(srcA, dstA, siA1, siA2)
        gfire(srcA, dstA, sixA, divA, gixA, asvA, advA, rowsA, sA1, sA2, sA3)
        idx_fire(jnp.minimum(g0 + 3, nb1), srcB, dstB, siB1, siB2)
        gwait(asvB, advB, rowsB, sB1, sB2, sB3)
        consume(divB, asvB, advB, exvB, rowsB)
        return 0

    lax.fori_loop(0, _NBLK // 2, it, 0)
    gwait(asvA, advA, rowsA, sA1, sA2, sA3)
    idx_wait(srcB, dstB, siB1, siB2)
    plsc.subcore_barrier()
    pltpu.sync_copy(accS.at[pl.ds(r0, _STRIPE)], numP.at[c, pl.ds(r0, _STRIPE)])
    pltpu.sync_copy(denS.at[pl.ds(r0, _STRIPE)], denP.at[c, pl.ds(r0, _STRIPE)])


def _pool_body(h3, bnd, psum, pmax, bndv, rbuf, rowb, osum, omax, sem):
    c = lax.axis_index("c")
    s = lax.axis_index("s")
    wid = c * 16 + s
    pltpu.sync_copy(bnd, bndv)
    iota = lax.iota(_I32, 16)

    def bval(t):
        blkoff = (t // 16) * 16
        vec = bndv[pl.ds(blkoff, 16)]
        return jnp.sum(jnp.where(iota == (t - blkoff), vec, 0))

    zf = jnp.zeros((16,), _F32)
    neg = jnp.full((16,), _NEG, _F32)
    for j in range(4):
        g = wid * 4 + j
        st = bval(g)
        en = bval(g + 1)
        cnt = en - st
        nfull = cnt // 16

        def fb(i, car):
            pltpu.sync_copy(h3.at[pl.ds(st + i * 16, 16)], rbuf)
            ss = list(car[:4])
            mm = list(car[4:])
            for r in range(16):
                for q in range(4):
                    v = rbuf[r, pl.ds(16 * q, 16)]
                    ss[q] = ss[q] + v
                    mm[q] = jnp.maximum(mm[q], v)
            return (*ss, *mm)

        car = lax.fori_loop(0, nfull, fb, (zf, zf, zf, zf, neg, neg, neg, neg))

        def tb(i, car2):
            pltpu.sync_copy(h3.at[pl.ds(st + nfull * 16 + i, 1)], rowb)
            ss = list(car2[:4])
            mm = list(car2[4:])
            for q in range(4):
                v = rowb[0, pl.ds(16 * q, 16)]
                ss[q] = ss[q] + v
                mm[q] = jnp.maximum(mm[q], v)
            return (*ss, *mm)

        car = lax.fori_loop(0, cnt - nfull * 16, tb, car)
        for q in range(4):
            osum[0, pl.ds(16 * q, 16)] = car[q]
            omax[0, pl.ds(16 * q, 16)] = car[4 + q]
        pltpu.sync_copy(osum, psum.at[pl.ds(g, 1)])
        pltpu.sync_copy(omax, pmax.at[pl.ds(g, 1)])


def _prep_tail(i, hout, wn_ref, ans_ref, adn_ref,
               hwn_ref, asn_ref, adn_out, mxs_ref, mxd_ref):
    hwn = jnp.dot(hout, wn_ref[...], preferred_element_type=_F32)
    hwn_ref[...] = jnp.stack([hwn[:, :32], hwn[:, 32:]], axis=0)
    a_s = jnp.sum(hwn * ans_ref[...], axis=1)
    a_d = jnp.sum(hwn * adn_ref[...], axis=1)
    asn_ref[...] = a_s.reshape(1, 1, _BN)
    adn_out[...] = a_d.reshape(1, 1, _BN)
    ms_prev = jnp.where(i == 0, _NEG, mxs_ref[...][0, 0])
    md_prev = jnp.where(i == 0, _NEG, mxd_ref[...][0, 0])
    mxs_ref[...] = jnp.maximum(ms_prev, jnp.max(a_s)).reshape(1, 1)
    mxd_ref[...] = jnp.maximum(md_prev, jnp.max(a_d)).reshape(1, 1)


def _tab_body(tab_ref, w_ref, a1s_ref, a1d_ref, o_ref):
    t = jnp.dot(tab_ref[...], w_ref[...], preferred_element_type=_F32)
    wa = jnp.sum(t * a1s_ref[...], axis=1, keepdims=True)
    wd = jnp.sum(t * a1d_ref[...], axis=1, keepdims=True)
    o_ref[...] = jnp.concatenate(
        [t, wa, wd, jnp.zeros((900, 14), _F32)], axis=1)


def _stats_body(asad_ref, batch_ref, asn_ref, adn_out, mxs_ref, mxd_ref,
                cnt_ref):
    i = pl.program_id(0)
    a_s = asad_ref[...][:, 0]
    a_d = asad_ref[...][:, 1]
    asn_ref[...] = a_s.reshape(1, 1, _BN)
    adn_out[...] = a_d.reshape(1, 1, _BN)
    ms_prev = jnp.where(i == 0, _NEG, mxs_ref[...][0, 0])
    md_prev = jnp.where(i == 0, _NEG, mxd_ref[...][0, 0])
    mxs_ref[...] = jnp.maximum(ms_prev, jnp.max(a_s)).reshape(1, 1)
    mxd_ref[...] = jnp.maximum(md_prev, jnp.max(a_d)).reshape(1, 1)
    b = batch_ref[...].reshape(_BN)
    oh = (b[:, None] == lax.broadcasted_iota(_I32, (_BN, _G), 1)).astype(_F32)
    c_prev = jnp.where(i == 0, jnp.zeros((1, _G), _F32), cnt_ref[...])
    cnt_ref[...] = c_prev + jnp.sum(oh, axis=0, keepdims=True)


def _combine(num0_ref, num1_ref, den_ref, hws0_ref, hws1_ref, as_ref, ad_ref,
             mxs_ref, mxd_ref, b_ref):
    mx = mxs_ref[...][0, 0] + mxd_ref[...][0, 0]
    m = jnp.maximum(mx, 0.2 * mx)
    t = as_ref[...].reshape(_BN) + ad_ref[...].reshape(_BN)
    e = jnp.where(t >= 0.0, t, 0.2 * t)
    exs = jnp.exp(e - m)
    hw = jnp.concatenate([hws0_ref[0], hws1_ref[0]], axis=1)
    num = jnp.concatenate([num0_ref[0], num1_ref[0]], axis=1)
    numv = num + exs[:, None] * hw
    denv = den_ref[...].reshape(_BN) + exs + 1e-16
    return jnp.maximum(numv / denv[:, None] + b_ref[...], 0.0)


def _cp_body(num0_ref, num1_ref, den_ref, hws0_ref, hws1_ref, as_ref, ad_ref,
             mxs_ref, mxd_ref, b_ref, wn_ref, ans_ref, adn_ref,
             hwn_ref, asn_ref, adn_out, mxs_out, mxd_out):
    i = pl.program_id(0)
    hout = _combine(num0_ref, num1_ref, den_ref, hws0_ref, hws1_ref, as_ref,
                    ad_ref, mxs_ref, mxd_ref, b_ref)
    _prep_tail(i, hout, wn_ref, ans_ref, adn_ref,
               hwn_ref, asn_ref, adn_out, mxs_out, mxd_out)


def _comb3_body(num0_ref, num1_ref, den_ref, hws0_ref, hws1_ref, as_ref,
                ad_ref, mxs_ref, mxd_ref, b_ref, cnt_ref, h3_ref, bnd_ref):
    h3_ref[...] = _combine(num0_ref, num1_ref, den_ref, hws0_ref, hws1_ref,
                           as_ref, ad_ref, mxs_ref, mxd_ref, b_ref)
    cnts = cnt_ref[...]
    ii = lax.broadcasted_iota(_I32, (_G, _G), 0)
    jj = lax.broadcasted_iota(_I32, (_G, _G), 1)
    tri = (ii <= jj).astype(_F32)
    cs = jnp.dot(cnts, tri, preferred_element_type=_F32)
    excl = (cs - cnts).astype(_I32)
    bnd_ref[...] = jnp.concatenate(
        [excl, jnp.full((1, 16), _N, _I32)], axis=1)


def _head_body(psum_ref, pmax_ref, cnt_ref, wl_ref, bl_ref, o_ref):
    cc = cnt_ref[...].reshape(_G, 1)
    ps = psum_ref[...]
    mean = ps / jnp.maximum(cc, 1.0)
    mx = jnp.where(cc > 0, pmax_ref[...], 0.0)
    pooled = jnp.concatenate([mean, ps, mx], axis=1)
    o_ref[...] = jnp.dot(pooled, wl_ref[...],
                         preferred_element_type=_F32) + bl_ref[...]


def _sc_mesh():
    return plsc.VectorSubcoreMesh(core_axis_name="c", subcore_axis_name="s")


def _emb_call(xf, taba, offs):
    kfn = pl.kernel(
        _emb_body,
        out_type=[jax.ShapeDtypeStruct((2, _PH0, 32), _F32),
                  jax.ShapeDtypeStruct((_PH0, 16), _F32)],
        mesh=_sc_mesh(),
        scratch_types=[
            pltpu.VMEM((288,), _I32),
            pltpu.VMEM((288,), _I32),
            pltpu.VMEM((288,), _I32),
            pltpu.VMEM((288, 80), _F32),
            pltpu.VMEM((288,), _I32),
            pltpu.VMEM((288,), _I32),
            pltpu.VMEM((288, 80), _F32),
            pltpu.VMEM((32, 80), _F32),
            pltpu.SemaphoreType.DMA,
            pltpu.SemaphoreType.DMA,
            pltpu.SemaphoreType.DMA,
            pltpu.SemaphoreType.DMA,
            pltpu.SemaphoreType.DMA,
            pltpu.SemaphoreType.DMA,
        ],
        compiler_params=pltpu.CompilerParams(use_tc_tiling_on_sc=False),
    )
    return kfn(xf, taba, offs)


def _tab_call(tabf, w1, a1s, a1d):
    return pl.pallas_call(
        _tab_body,
        out_shape=jax.ShapeDtypeStruct((900, 80), _F32),
    )(tabf, w1, a1s, a1d)


def _edge_call(hws, as3, ad3, mxs, mxd, srcP, dstP):
    asf = as3.reshape(_N)
    adf = ad3.reshape(_N)
    hwsf = hws.reshape(2 * _PH0, 32)
    mx = mxs[0, 0] + mxd[0, 0]
    m = jnp.maximum(mx, 0.2 * mx)
    mv = jnp.full((16,), 1.0, _F32) * m
    kfn = pl.kernel(
        _edge_body,
        out_type=[jax.ShapeDtypeStruct((2, _PN, 32), _F32),
                  jax.ShapeDtypeStruct((2, _PN), _F32)],
        mesh=_sc_mesh(),
        scratch_types=(
            [pltpu.VMEM_SHARED((_PN, 32), _F32),
             pltpu.VMEM_SHARED((_PN,), _F32),
             pltpu.VMEM((_EB, 32), _F32),
             pltpu.VMEM((_STRIPE,), _F32),
             pltpu.VMEM((16,), _F32)]
            + 2 * ([pltpu.VMEM((_EB,), _I32)] * 5
                   + [pltpu.VMEM((_EB,), _F32)] * 3
                   + [pltpu.VMEM((_EB, 32), _F32)])
            + [pltpu.SemaphoreType.DMA] * 10
        ),
        compiler_params=pltpu.CompilerParams(use_tc_tiling_on_sc=False),
    )
    return kfn(hwsf, asf, adf, mv, srcP, dstP)


def _pool_call(h3, bnd):
    kfn = pl.kernel(
        _pool_body,
        out_type=[jax.ShapeDtypeStruct((_G, 64), _F32),
                  jax.ShapeDtypeStruct((_G, 64), _F32)],
        mesh=_sc_mesh(),
        scratch_types=[
            pltpu.VMEM((144,), _I32),
            pltpu.VMEM((16, 64), _F32),
            pltpu.VMEM((1, 64), _F32),
            pltpu.VMEM((1, 64), _F32),
            pltpu.VMEM((1, 64), _F32),
            pltpu.SemaphoreType.DMA,
        ],
        compiler_params=pltpu.CompilerParams(use_tc_tiling_on_sc=False,
                                             needs_layout_passes=False),
    )
    return kfn(h3, bnd)


def _full(shape):
    return pl.BlockSpec(shape, lambda b: tuple(0 for _ in shape))


def _stats_call(asadE, batch3):
    return pl.pallas_call(
        _stats_body,
        grid=(_GRID,),
        in_specs=[
            pl.BlockSpec((_BN, 16), lambda b: (b, 0)),
            pl.BlockSpec((1, 1, _BN), lambda b: (b, 0, 0)),
        ],
        out_specs=[
            pl.BlockSpec((1, 1, _BN), lambda b: (b, 0, 0)),
            pl.BlockSpec((1, 1, _BN), lambda b: (b, 0, 0)),
            _full((1, 1)),
            _full((1, 1)),
            _full((1, _G)),
        ],
        out_shape=[
            jax.ShapeDtypeStruct((_GRID, 1, _BN), _F32),
            jax.ShapeDtypeStruct((_GRID, 1, _BN), _F32),
            jax.ShapeDtypeStruct((1, 1), _F32),
            jax.ShapeDtypeStruct((1, 1), _F32),
            jax.ShapeDtypeStruct((1, _G), _F32),
        ],
    )(asadE, batch3)


_CP_IN_SPECS = [
    pl.BlockSpec((1, _BN, 32), lambda b: (0, b, 0)),
    pl.BlockSpec((1, _BN, 32), lambda b: (1, b, 0)),
    pl.BlockSpec((1, 1, _BN), lambda b: (b, 0, 0)),
    pl.BlockSpec((1, _BN, 32), lambda b: (0, b, 0)),
    pl.BlockSpec((1, _BN, 32), lambda b: (1, b, 0)),
    pl.BlockSpec((1, 1, _BN), lambda b: (b, 0, 0)),
    pl.BlockSpec((1, 1, _BN), lambda b: (b, 0, 0)),
    _full((1, 1)),
    _full((1, 1)),
    _full((1, 64)),
]


def _cp_call(numP, denP, hws, as3, ad3, mxs, mxd, bb, wn, ans, adn):
    den3 = denP[0, :_N].reshape(_GRID, 1, _BN)
    return pl.pallas_call(
        _cp_body,
        grid=(_GRID,),
        in_specs=_CP_IN_SPECS + [
            _full((64, 64)),
            _full((1, 64)),
            _full((1, 64)),
        ],
        out_specs=[
            pl.BlockSpec((2, _BN, 32), lambda b: (0, b, 0)),
            pl.BlockSpec((1, 1, _BN), lambda b: (b, 0, 0)),
            pl.BlockSpec((1, 1, _BN), lambda b: (b, 0, 0)),
            _full((1, 1)),
            _full((1, 1)),
        ],
        out_shape=[
            jax.ShapeDtypeStruct((2, _PH0, 32), _F32),
            jax.ShapeDtypeStruct((_GRID, 1, _BN), _F32),
            jax.ShapeDtypeStruct((_GRID, 1, _BN), _F32),
            jax.ShapeDtypeStruct((1, 1), _F32),
            jax.ShapeDtypeStruct((1, 1), _F32),
        ],
    )(numP, numP, den3, hws, hws, as3, ad3, mxs, mxd, bb.reshape(1, 64),
      wn, ans.reshape(1, 64), adn.reshape(1, 64))


def _comb3_call(numP, denP, hws, as3, ad3, mxs, mxd, b3, counts):
    den3 = denP[0, :_N].reshape(_GRID, 1, _BN)
    return pl.pallas_call(
        _comb3_body,
        grid=(_GRID,),
        in_specs=_CP_IN_SPECS + [_full((1, _G))],
        out_specs=[
            pl.BlockSpec((_BN, 64), lambda b: (b, 0)),
            _full((1, 144)),
        ],
        out_shape=[
            jax.ShapeDtypeStruct((_N, 64), _F32),
            jax.ShapeDtypeStruct((1, 144), _I32),
        ],
    )(numP, numP, den3, hws, hws, as3, ad3, mxs, mxd, b3.reshape(1, 64),
      counts)


def _head_call(psum, pmax, counts, w_lin, b_lin):
    return pl.pallas_call(
        _head_body,
        out_shape=jax.ShapeDtypeStruct((_G, w_lin.shape[1]), _F32),
    )(psum, pmax, counts, w_lin, b_lin.reshape(1, -1))


def kernel(x, edge_index, batch, atom_emb, W1, a1_src, a1_dst, b1, W2, a2_src,
           a2_dst, b2, W3, a3_src, a3_dst, b3, W_lin, b_lin):
    x = x.astype(_I32)
    xf = jnp.pad(x.reshape(-1), (0, _PH0 * 9 - _N * 9))
    tabf = atom_emb.reshape(900, 128)
    offs = jnp.tile(jnp.arange(9, dtype=_I32) * 100, 32)
    taba = _tab_call(tabf, W1, a1_src.reshape(1, 64), a1_dst.reshape(1, 64))
    hws, asadE = _emb_call(xf, taba, offs)

    srcP = jnp.concatenate(
        [edge_index[0].astype(_I32), jnp.zeros((_EPAD - _E,), _I32)])
    dstP = jnp.concatenate(
        [edge_index[1].astype(_I32), jnp.full((_EPAD - _E,), _N, _I32)])
    batch3 = batch.astype(_I32).reshape(_GRID, 1, _BN)

    as3, ad3, mxs, mxd, counts = _stats_call(asadE, batch3)

    for (bb, wn, ans, adn) in ((b1, W2, a2_src, a2_dst),
                               (b2, W3, a3_src, a3_dst)):
        numP, denP = _edge_call(hws, as3, ad3, mxs, mxd, srcP, dstP)
        hws, as3, ad3, mxs, mxd = _cp_call(
            numP, denP, hws, as3, ad3, mxs, mxd, bb, wn, ans, adn)

    numP, denP = _edge_call(hws, as3, ad3, mxs, mxd, srcP, dstP)
    h3, bnd = _comb3_call(numP, denP, hws, as3, ad3, mxs, mxd, b3, counts)

    psum, pmax = _pool_call(h3, bnd.reshape(144))
    return _head_call(psum, pmax, counts, W_lin, b_lin)


# speedup vs baseline: 48.0997x; 1.0118x over previous
import jax
import jax.numpy as jnp
from jax import lax
from jax.experimental import pallas as pl
from jax.experimental.pallas import tpu as pltpu
from jax.experimental.pallas import tpu_sc as plsc

_N = 50000
_E = 800000
_G = 128
_EB = 128
_NBLK = 392
_NE_T = _EB * _NBLK
_EPAD = 16 * _NE_T
_PN = 51200
_STRIPE = _PN // 16
_PH0 = 50176
_BN = 1000
_GRID = _N // _BN
_F32 = jnp.float32
_I32 = jnp.int32
_NEG = -3.4e38


def _emb_body(xf, taba, offs, hws, asad, ov,
              xvA, idxA, rowsA, xvB, idxB, rowsB, hout,
              sa1, sa2, sa3, sb1, sb2, sb3):
    c = lax.axis_index("c")
    s = lax.axis_index("s")
    wid = c * 16 + s
    pltpu.sync_copy(offs, ov)

    def fire(b, xv, idxv, rows, s1, s2, s3):
        node0 = wid * 1568 + b * 32
        pltpu.sync_copy(xf.at[pl.ds(node0 * 9, 288)], xv)
        for q in range(18):
            idxv[pl.ds(16 * q, 16)] = (xv[pl.ds(16 * q, 16)]
                                       + ov[pl.ds(16 * q, 16)])
        pltpu.async_copy(taba.at[idxv.at[pl.ds(0, 128)]],
                         rows.at[pl.ds(0, 128)], s1)
        pltpu.async_copy(taba.at[idxv.at[pl.ds(128, 128)]],
                         rows.at[pl.ds(128, 128)], s2)
        pltpu.async_copy(taba.at[idxv.at[pl.ds(256, 32)]],
                         rows.at[pl.ds(256, 32)], s3)

    def consume(b, rows, s1, s2, s3):
        pltpu.make_async_copy(taba.at[pl.ds(0, 128)],
                             rows.at[pl.ds(0, 128)], s1).wait()
        pltpu.make_async_copy(taba.at[pl.ds(0, 128)],
                             rows.at[pl.ds(128, 128)], s2).wait()
        pltpu.make_async_copy(taba.at[pl.ds(0, 32)],
                             rows.at[pl.ds(256, 32)], s3).wait()

        def nrow(r, _2):
            rr = r * 9
            for q in range(5):
                acc = rows[rr, pl.ds(16 * q, 16)]
                for t in range(1, 9):
                    acc = acc + rows[rr + t, pl.ds(16 * q, 16)]
                hout[r, pl.ds(16 * q, 16)] = acc
            return 0

        lax.fori_loop(0, 32, nrow, 0)
        node0 = wid * 1568 + b * 32
        pltpu.sync_copy(hout.at[:, pl.ds(0, 32)], hws.at[0, pl.ds(node0, 32)])
        pltpu.sync_copy(hout.at[:, pl.ds(32, 32)], hws.at[1, pl.ds(node0, 32)])
        pltpu.sync_copy(hout.at[:, pl.ds(64, 16)], asad.at[pl.ds(node0, 32)])

    fire(0, xvA, idxA, rowsA, sa1, sa2, sa3)

    def it(j, _):
        fire(2 * j + 1, xvB, idxB, rowsB, sb1, sb2, sb3)
        consume(2 * j, rowsA, sa1, sa2, sa3)
        fire(2 * j + 2, xvA, idxA, rowsA, sa1, sa2, sa3)
        consume(2 * j + 1, rowsB, sb1, sb2, sb3)
        return 0

    lax.fori_loop(0, 24, it, 0)
    consume(48, rowsA, sa1, sa2, sa3)


def _edge_body(hwsf, asf, adf, mv, srcP, dstP, numP, denP,
               accS, denS, z32, zden, mvv,
               srcA, dstA, sixA, divA, gixA, dnxA, asvA, advA, exvA, rowsA,
               srcB, dstB, sixB, divB, gixB, dnxB, asvB, advB, exvB, rowsB,
               sA1, sA2, sA3, sB1, sB2, sB3, siA1, siA2, siB1, siB2,
               sdA, sdB):
    c = lax.axis_index("c")
    s = lax.axis_index("s")
    zf = jnp.zeros((16,), _F32)

    def z1(i, _):
        z32[i, pl.ds(0, 16)] = zf
        z32[i, pl.ds(16, 16)] = zf
        return 0

    lax.fori_loop(0, _EB, z1, 0)

    def z2(i, _):
        zden[pl.ds(i * 16, 16)] = zf
        return 0

    lax.fori_loop(0, _STRIPE // 16, z2, 0)
    r0 = s * _STRIPE
    for k in range(_STRIPE // _EB):
        pltpu.sync_copy(z32, accS.at[pl.ds(r0 + k * _EB, _EB)])
    pltpu.sync_copy(zden, denS.at[pl.ds(r0, _STRIPE)])
    pltpu.sync_copy(mv, mvv)
    plsc.subcore_barrier()

    coff = c * _PH0
    e0 = s * _NE_T
    nb1 = _NBLK - 1

    def idx_fire(b, sv, dv, s1, s2):
        base = e0 + b * _EB
        pltpu.async_copy(srcP.at[pl.ds(base, _EB)], sv, s1)
        pltpu.async_copy(dstP.at[pl.ds(base, _EB)], dv, s2)

    def idx_wait(sv, dv, s1, s2):
        pltpu.make_async_copy(srcP.at[pl.ds(0, _EB)], sv, s1).wait()
        pltpu.make_async_copy(dstP.at[pl.ds(0, _EB)], dv, s2).wait()

    def gfire(sv, dv, six, div, gix, asv, adv, rows, s1, s2, s3):
        for q in range(8):
            sq = sv[pl.ds(16 * q, 16)]
            six[pl.ds(16 * q, 16)] = sq
            gix[pl.ds(16 * q, 16)] = sq + coff
            div[pl.ds(16 * q, 16)] = dv[pl.ds(16 * q, 16)]
        pltpu.async_copy(asf.at[six], asv, s1)
        pltpu.async_copy(adf.at[div], adv, s2)
        pltpu.async_copy(hwsf.at[gix], rows, s3)

    def gwait(asv, adv, rows, s1, s2, s3):
        pltpu.make_async_copy(asf.at[pl.ds(0, _EB)], asv, s1).wait()
        pltpu.make_async_copy(adf.at[pl.ds(0, _EB)], adv, s2).wait()
        pltpu.make_async_copy(hwsf.at[pl.ds(0, _EB)], rows, s3).wait()

    def consume(div, dnx, asv, adv, exv, rows, sd):
        pltpu.make_async_copy(asf.at[pl.ds(0, _EB)], exv, sd).wait()
        mvec = mvv[pl.ds(0, 16)]
        for q in range(8):
            t = asv[pl.ds(16 * q, 16)] + adv[pl.ds(16 * q, 16)]
            e = jnp.where(t >= 0.0, t, 0.2 * t)
            exv[pl.ds(16 * q, 16)] = jnp.exp(e - mvec)

        def scg(g, _2):
            exvec = exv[pl.ds(g * 16, 16)]
            for r in range(16):
                sp = lax.gather(
                    exvec, jnp.full((16, 1), r, _I32),
                    lax.GatherDimensionNumbers(
                        offset_dims=(), collapsed_slice_dims=(0,),
                        start_index_map=(0,)),
                    slice_sizes=(1,),
                    mode=lax.GatherScatterMode.PROMISE_IN_BOUNDS)
                rr = g * 16 + r
                rows[rr, pl.ds(0, 16)] = rows[rr, pl.ds(0, 16)] * sp
                rows[rr, pl.ds(16, 16)] = rows[rr, pl.ds(16, 16)] * sp
            return 0

        lax.fori_loop(0, _EB // 16, scg, 0)
        pltpu.sync_copy(rows, accS.at[div], add=True)
        for q in range(8):
            dnx[pl.ds(16 * q, 16)] = div[pl.ds(16 * q, 16)]
        pltpu.async_copy(exv, denS.at[dnx], sd, add=True)

    pltpu.sync_copy(srcP.at[pl.ds(e0, _EB)], srcA)
    pltpu.sync_copy(dstP.at[pl.ds(e0, _EB)], dstA)
    gfire(srcA, dstA, sixA, divA, gixA, asvA, advA, rowsA, sA1, sA2, sA3)
    idx_fire(1, srcB, dstB, siB1, siB2)
    pltpu.async_copy(exvA, denS.at[pl.ds(_N + 432, _EB)], sdA)
    pltpu.async_copy(exvB, denS.at[pl.ds(_N + 432, _EB)], sdB)

    def it(j, _):
        g0 = 2 * j
        idx_wait(srcB, dstB, siB1, siB2)
        gfire(srcB, dstB, sixB, divB, gixB, asvB, advB, rowsB, sB1, sB2, sB3)
        idx_fire(jnp.minimum(g0 + 2, nb1), srcA, dstA, siA1, siA2)
        gwait(asvA, advA, rowsA, sA1, sA2, sA3)
        consume(divA, dnxA, asvA, advA, exvA, rowsA, sdA)
        idx_wait(srcA, dstA, siA1, siA2)
        gfire(srcA, dstA, sixA, divA, gixA, asvA, advA, rowsA, sA1, sA2, sA3)
        idx_fire(jnp.minimum(g0 + 3, nb1), srcB, dstB, siB1, siB2)
        gwait(asvB, advB, rowsB, sB1, sB2, sB3)
        consume(divB, dnxB, asvB, advB, exvB, rowsB, sdB)
        return 0

    lax.fori_loop(0, _NBLK // 2, it, 0)
    gwait(asvA, advA, rowsA, sA1, sA2, sA3)
    idx_wait(srcB, dstB, siB1, siB2)
    pltpu.make_async_copy(asf.at[pl.ds(0, _EB)], exvA, sdA).wait()
    pltpu.make_async_copy(asf.at[pl.ds(0, _EB)], exvB, sdB).wait()
    plsc.subcore_barrier()
    pltpu.sync_copy(accS.at[pl.ds(r0, _STRIPE)], numP.at[c, pl.ds(r0, _STRIPE)])
    pltpu.sync_copy(denS.at[pl.ds(r0, _STRIPE)], denP.at[c, pl.ds(r0, _STRIPE)])


def _pool_body(h3, bnd, psum, pmax, bndv, rbuf, rowb, osum, omax, sem):
    c = lax.axis_index("c")
    s = lax.axis_index("s")
    wid = c * 16 + s
    pltpu.sync_copy(bnd, bndv)
    iota = lax.iota(_I32, 16)

    def bval(t):
        blkoff = (t // 16) * 16
        vec = bndv[pl.ds(blkoff, 16)]
        return jnp.sum(jnp.where(iota == (t - blkoff), vec, 0))

    zf = jnp.zeros((16,), _F32)
    neg = jnp.full((16,), _NEG, _F32)
    for j in range(4):
        g = wid * 4 + j
        st = bval(g)
        en = bval(g + 1)
        cnt = en - st
        nfull = cnt // 16

        def fb(i, car):
            pltpu.sync_copy(h3.at[pl.ds(st + i * 16, 16)], rbuf)
            ss = list(car[:4])
            mm = list(car[4:])
            for r in range(16):
                for q in range(4):
                    v = rbuf[r, pl.ds(16 * q, 16)]
                    ss[q] = ss[q] + v
                    mm[q] = jnp.maximum(mm[q], v)
            return (*ss, *mm)

        car = lax.fori_loop(0, nfull, fb, (zf, zf, zf, zf, neg, neg, neg, neg))

        def tb(i, car2):
            pltpu.sync_copy(h3.at[pl.ds(st + nfull * 16 + i, 1)], rowb)
            ss = list(car2[:4])
            mm = list(car2[4:])
            for q in range(4):
                v = rowb[0, pl.ds(16 * q, 16)]
                ss[q] = ss[q] + v
                mm[q] = jnp.maximum(mm[q], v)
            return (*ss, *mm)

        car = lax.fori_loop(0, cnt - nfull * 16, tb, car)
        for q in range(4):
            osum[0, pl.ds(16 * q, 16)] = car[q]
            omax[0, pl.ds(16 * q, 16)] = car[4 + q]
        pltpu.sync_copy(osum, psum.at[pl.ds(g, 1)])
        pltpu.sync_copy(omax, pmax.at[pl.ds(g, 1)])


def _prep_tail(i, hout, wn_ref, ans_ref, adn_ref,
               hwn_ref, asn_ref, adn_out, mxs_ref, mxd_ref):
    hwn = jnp.dot(hout, wn_ref[...], preferred_element_type=_F32)
    hwn_ref[...] = jnp.stack([hwn[:, :32], hwn[:, 32:]], axis=0)
    a_s = jnp.sum(hwn * ans_ref[...], axis=1)
    a_d = jnp.sum(hwn * adn_ref[...], axis=1)
    asn_ref[...] = a_s.reshape(1, 1, _BN)
    adn_out[...] = a_d.reshape(1, 1, _BN)
    ms_prev = jnp.where(i == 0, _NEG, mxs_ref[...][0, 0])
    md_prev = jnp.where(i == 0, _NEG, mxd_ref[...][0, 0])
    mxs_ref[...] = jnp.maximum(ms_prev, jnp.max(a_s)).reshape(1, 1)
    mxd_ref[...] = jnp.maximum(md_prev, jnp.max(a_d)).reshape(1, 1)


def _tab_body(tab_ref, w_ref, a1s_ref, a1d_ref, o_ref):
    t = jnp.dot(tab_ref[...], w_ref[...], preferred_element_type=_F32)
    wa = jnp.sum(t * a1s_ref[...], axis=1, keepdims=True)
    wd = jnp.sum(t * a1d_ref[...], axis=1, keepdims=True)
    o_ref[...] = jnp.concatenate(
        [t, wa, wd, jnp.zeros((900, 14), _F32)], axis=1)


def _stats_body(asad_ref, batch_ref, asn_ref, adn_out, mxs_ref, mxd_ref,
                cnt_ref):
    i = pl.program_id(0)
    a_s = asad_ref[...][:, 0]
    a_d = asad_ref[...][:, 1]
    asn_ref[...] = a_s.reshape(1, 1, _BN)
    adn_out[...] = a_d.reshape(1, 1, _BN)
    ms_prev = jnp.where(i == 0, _NEG, mxs_ref[...][0, 0])
    md_prev = jnp.where(i == 0, _NEG, mxd_ref[...][0, 0])
    mxs_ref[...] = jnp.maximum(ms_prev, jnp.max(a_s)).reshape(1, 1)
    mxd_ref[...] = jnp.maximum(md_prev, jnp.max(a_d)).reshape(1, 1)
    b = batch_ref[...].reshape(_BN)
    oh = (b[:, None] == lax.broadcasted_iota(_I32, (_BN, _G), 1)).astype(_F32)
    c_prev = jnp.where(i == 0, jnp.zeros((1, _G), _F32), cnt_ref[...])
    cnt_ref[...] = c_prev + jnp.sum(oh, axis=0, keepdims=True)


def _combine(num0_ref, num1_ref, den_ref, hws0_ref, hws1_ref, as_ref, ad_ref,
             mxs_ref, mxd_ref, b_ref):
    mx = mxs_ref[...][0, 0] + mxd_ref[...][0, 0]
    m = jnp.maximum(mx, 0.2 * mx)
    t = as_ref[...].reshape(_BN) + ad_ref[...].reshape(_BN)
    e = jnp.where(t >= 0.0, t, 0.2 * t)
    exs = jnp.exp(e - m)
    hw = jnp.concatenate([hws0_ref[0], hws1_ref[0]], axis=1)
    num = jnp.concatenate([num0_ref[0], num1_ref[0]], axis=1)
    numv = num + exs[:, None] * hw
    denv = den_ref[...].reshape(_BN) + exs + 1e-16
    return jnp.maximum(numv / denv[:, None] + b_ref[...], 0.0)


def _cp_body(num0_ref, num1_ref, den_ref, hws0_ref, hws1_ref, as_ref, ad_ref,
             mxs_ref, mxd_ref, b_ref, wn_ref, ans_ref, adn_ref,
             hwn_ref, asn_ref, adn_out, mxs_out, mxd_out):
    i = pl.program_id(0)
    hout = _combine(num0_ref, num1_ref, den_ref, hws0_ref, hws1_ref, as_ref,
                    ad_ref, mxs_ref, mxd_ref, b_ref)
    _prep_tail(i, hout, wn_ref, ans_ref, adn_ref,
               hwn_ref, asn_ref, adn_out, mxs_out, mxd_out)


def _comb3_body(num0_ref, num1_ref, den_ref, hws0_ref, hws1_ref, as_ref,
                ad_ref, mxs_ref, mxd_ref, b_ref, cnt_ref, h3_ref, bnd_ref):
    h3_ref[...] = _combine(num0_ref, num1_ref, den_ref, hws0_ref, hws1_ref,
                           as_ref, ad_ref, mxs_ref, mxd_ref, b_ref)
    cnts = cnt_ref[...]
    ii = lax.broadcasted_iota(_I32, (_G, _G), 0)
    jj = lax.broadcasted_iota(_I32, (_G, _G), 1)
    tri = (ii <= jj).astype(_F32)
    cs = jnp.dot(cnts, tri, preferred_element_type=_F32)
    excl = (cs - cnts).astype(_I32)
    bnd_ref[...] = jnp.concatenate(
        [excl, jnp.full((1, 16), _N, _I32)], axis=1)


def _head_body(psum_ref, pmax_ref, cnt_ref, wl_ref, bl_ref, o_ref):
    cc = cnt_ref[...].reshape(_G, 1)
    ps = psum_ref[...]
    mean = ps / jnp.maximum(cc, 1.0)
    mx = jnp.where(cc > 0, pmax_ref[...], 0.0)
    pooled = jnp.concatenate([mean, ps, mx], axis=1)
    o_ref[...] = jnp.dot(pooled, wl_ref[...],
                         preferred_element_type=_F32) + bl_ref[...]


def _sc_mesh():
    return plsc.VectorSubcoreMesh(core_axis_name="c", subcore_axis_name="s")


def _emb_call(xf, taba, offs):
    kfn = pl.kernel(
        _emb_body,
        out_type=[jax.ShapeDtypeStruct((2, _PH0, 32), _F32),
                  jax.ShapeDtypeStruct((_PH0, 16), _F32)],
        mesh=_sc_mesh(),
        scratch_types=[
            pltpu.VMEM((288,), _I32),
            pltpu.VMEM((288,), _I32),
            pltpu.VMEM((288,), _I32),
            pltpu.VMEM((288, 80), _F32),
            pltpu.VMEM((288,), _I32),
            pltpu.VMEM((288,), _I32),
            pltpu.VMEM((288, 80), _F32),
            pltpu.VMEM((32, 80), _F32),
            pltpu.SemaphoreType.DMA,
            pltpu.SemaphoreType.DMA,
            pltpu.SemaphoreType.DMA,
            pltpu.SemaphoreType.DMA,
            pltpu.SemaphoreType.DMA,
            pltpu.SemaphoreType.DMA,
        ],
        compiler_params=pltpu.CompilerParams(use_tc_tiling_on_sc=False),
    )
    return kfn(xf, taba, offs)


def _tab_call(tabf, w1, a1s, a1d):
    return pl.pallas_call(
        _tab_body,
        out_shape=jax.ShapeDtypeStruct((900, 80), _F32),
    )(tabf, w1, a1s, a1d)


def _edge_call(hws, as3, ad3, mxs, mxd, srcP, dstP):
    asf = as3.reshape(_N)
    adf = ad3.reshape(_N)
    hwsf = hws.reshape(2 * _PH0, 32)
    mx = mxs[0, 0] + mxd[0, 0]
    m = jnp.maximum(mx, 0.2 * mx)
    mv = jnp.full((16,), 1.0, _F32) * m
    kfn = pl.kernel(
        _edge_body,
        out_type=[jax.ShapeDtypeStruct((2, _PN, 32), _F32),
                  jax.ShapeDtypeStruct((2, _PN), _F32)],
        mesh=_sc_mesh(),
        scratch_types=(
            [pltpu.VMEM_SHARED((_PN, 32), _F32),
             pltpu.VMEM_SHARED((_PN,), _F32),
             pltpu.VMEM((_EB, 32), _F32),
             pltpu.VMEM((_STRIPE,), _F32),
             pltpu.VMEM((16,), _F32)]
            + 2 * ([pltpu.VMEM((_EB,), _I32)] * 6
                   + [pltpu.VMEM((_EB,), _F32)] * 3
                   + [pltpu.VMEM((_EB, 32), _F32)])
            + [pltpu.SemaphoreType.DMA] * 12
        ),
        compiler_params=pltpu.CompilerParams(use_tc_tiling_on_sc=False),
    )
    return kfn(hwsf, asf, adf, mv, srcP, dstP)


def _pool_call(h3, bnd):
    kfn = pl.kernel(
        _pool_body,
        out_type=[jax.ShapeDtypeStruct((_G, 64), _F32),
                  jax.ShapeDtypeStruct((_G, 64), _F32)],
        mesh=_sc_mesh(),
        scratch_types=[
            pltpu.VMEM((144,), _I32),
            pltpu.VMEM((16, 64), _F32),
            pltpu.VMEM((1, 64), _F32),
            pltpu.VMEM((1, 64), _F32),
            pltpu.VMEM((1, 64), _F32),
            pltpu.SemaphoreType.DMA,
        ],
        compiler_params=pltpu.CompilerParams(use_tc_tiling_on_sc=False,
                                             needs_layout_passes=False),
    )
    return kfn(h3, bnd)


def _full(shape):
    return pl.BlockSpec(shape, lambda b: tuple(0 for _ in shape))


def _stats_call(asadE, batch3):
    return pl.pallas_call(
        _stats_body,
        grid=(_GRID,),
        in_specs=[
            pl.BlockSpec((_BN, 16), lambda b: (b, 0)),
            pl.BlockSpec((1, 1, _BN), lambda b: (b, 0, 0)),
        ],
        out_specs=[
            pl.BlockSpec((1, 1, _BN), lambda b: (b, 0, 0)),
            pl.BlockSpec((1, 1, _BN), lambda b: (b, 0, 0)),
            _full((1, 1)),
            _full((1, 1)),
            _full((1, _G)),
        ],
        out_shape=[
            jax.ShapeDtypeStruct((_GRID, 1, _BN), _F32),
            jax.ShapeDtypeStruct((_GRID, 1, _BN), _F32),
            jax.ShapeDtypeStruct((1, 1), _F32),
            jax.ShapeDtypeStruct((1, 1), _F32),
            jax.ShapeDtypeStruct((1, _G), _F32),
        ],
    )(asadE, batch3)


_CP_IN_SPECS = [
    pl.BlockSpec((1, _BN, 32), lambda b: (0, b, 0)),
    pl.BlockSpec((1, _BN, 32), lambda b: (1, b, 0)),
    pl.BlockSpec((1, 1, _BN), lambda b: (b, 0, 0)),
    pl.BlockSpec((1, _BN, 32), lambda b: (0, b, 0)),
    pl.BlockSpec((1, _BN, 32), lambda b: (1, b, 0)),
    pl.BlockSpec((1, 1, _BN), lambda b: (b, 0, 0)),
    pl.BlockSpec((1, 1, _BN), lambda b: (b, 0, 0)),
    _full((1, 1)),
    _full((1, 1)),
    _full((1, 64)),
]


def _cp_call(numP, denP, hws, as3, ad3, mxs, mxd, bb, wn, ans, adn):
    den3 = denP[0, :_N].reshape(_GRID, 1, _BN)
    return pl.pallas_call(
        _cp_body,
        grid=(_GRID,),
        in_specs=_CP_IN_SPECS + [
            _full((64, 64)),
            _full((1, 64)),
            _full((1, 64)),
        ],
        out_specs=[
            pl.BlockSpec((2, _BN, 32), lambda b: (0, b, 0)),
            pl.BlockSpec((1, 1, _BN), lambda b: (b, 0, 0)),
            pl.BlockSpec((1, 1, _BN), lambda b: (b, 0, 0)),
            _full((1, 1)),
            _full((1, 1)),
        ],
        out_shape=[
            jax.ShapeDtypeStruct((2, _PH0, 32), _F32),
            jax.ShapeDtypeStruct((_GRID, 1, _BN), _F32),
            jax.ShapeDtypeStruct((_GRID, 1, _BN), _F32),
            jax.ShapeDtypeStruct((1, 1), _F32),
            jax.ShapeDtypeStruct((1, 1), _F32),
        ],
    )(numP, numP, den3, hws, hws, as3, ad3, mxs, mxd, bb.reshape(1, 64),
      wn, ans.reshape(1, 64), adn.reshape(1, 64))


def _comb3_call(numP, denP, hws, as3, ad3, mxs, mxd, b3, counts):
    den3 = denP[0, :_N].reshape(_GRID, 1, _BN)
    return pl.pallas_call(
        _comb3_body,
        grid=(_GRID,),
        in_specs=_CP_IN_SPECS + [_full((1, _G))],
        out_specs=[
            pl.BlockSpec((_BN, 64), lambda b: (b, 0)),
            _full((1, 144)),
        ],
        out_shape=[
            jax.ShapeDtypeStruct((_N, 64), _F32),
            jax.ShapeDtypeStruct((1, 144), _I32),
        ],
    )(numP, numP, den3, hws, hws, as3, ad3, mxs, mxd, b3.reshape(1, 64),
      counts)


def _head_call(psum, pmax, counts, w_lin, b_lin):
    return pl.pallas_call(
        _head_body,
        out_shape=jax.ShapeDtypeStruct((_G, w_lin.shape[1]), _F32),
    )(psum, pmax, counts, w_lin, b_lin.reshape(1, -1))


def kernel(x, edge_index, batch, atom_emb, W1, a1_src, a1_dst, b1, W2, a2_src,
           a2_dst, b2, W3, a3_src, a3_dst, b3, W_lin, b_lin):
    x = x.astype(_I32)
    xf = jnp.pad(x.reshape(-1), (0, _PH0 * 9 - _N * 9))
    tabf = atom_emb.reshape(900, 128)
    offs = jnp.tile(jnp.arange(9, dtype=_I32) * 100, 32)
    taba = _tab_call(tabf, W1, a1_src.reshape(1, 64), a1_dst.reshape(1, 64))
    hws, asadE = _emb_call(xf, taba, offs)

    srcP = jnp.concatenate(
        [edge_index[0].astype(_I32), jnp.zeros((_EPAD - _E,), _I32)])
    dstP = jnp.concatenate(
        [edge_index[1].astype(_I32), jnp.full((_EPAD - _E,), _N, _I32)])
    batch3 = batch.astype(_I32).reshape(_GRID, 1, _BN)

    as3, ad3, mxs, mxd, counts = _stats_call(asadE, batch3)

    for (bb, wn, ans, adn) in ((b1, W2, a2_src, a2_dst),
                               (b2, W3, a3_src, a3_dst)):
        numP, denP = _edge_call(hws, as3, ad3, mxs, mxd, srcP, dstP)
        hws, as3, ad3, mxs, mxd = _cp_call(
            numP, denP, hws, as3, ad3, mxs, mxd, bb, wn, ans, adn)

    numP, denP = _edge_call(hws, as3, ad3, mxs, mxd, srcP, dstP)
    h3, bnd = _comb3_call(numP, denP, hws, as3, ad3, mxs, mxd, b3, counts)

    psum, pmax = _pool_call(h3, bnd.reshape(144))
    return _head_call(psum, pmax, counts, W_lin, b_lin)


# speedup vs baseline: 49.0171x; 1.0191x over previous
import jax
import jax.numpy as jnp
from jax import lax
from jax.experimental import pallas as pl
from jax.experimental.pallas import tpu as pltpu
from jax.experimental.pallas import tpu_sc as plsc

_N = 50000
_E = 800000
_G = 128
_EB = 128
_NBLK = 392
_NE_T = _EB * _NBLK
_EPAD = 16 * _NE_T
_PN = 51200
_STRIPE = _PN // 16
_PH0 = 50176
_BN = 2000
_GRID = _N // _BN
_F32 = jnp.float32
_I32 = jnp.int32
_NEG = -3.4e38


def _emb_body(xf, taba, offs, hws, asad, ov,
              xvA, idxA, rowsA, xvB, idxB, rowsB, hout,
              sa1, sa2, sa3, sb1, sb2, sb3):
    c = lax.axis_index("c")
    s = lax.axis_index("s")
    wid = c * 16 + s
    pltpu.sync_copy(offs, ov)

    def fire(b, xv, idxv, rows, s1, s2, s3):
        node0 = wid * 1568 + b * 32
        pltpu.sync_copy(xf.at[pl.ds(node0 * 9, 288)], xv)
        for q in range(18):
            idxv[pl.ds(16 * q, 16)] = (xv[pl.ds(16 * q, 16)]
                                       + ov[pl.ds(16 * q, 16)])
        pltpu.async_copy(taba.at[idxv.at[pl.ds(0, 128)]],
                         rows.at[pl.ds(0, 128)], s1)
        pltpu.async_copy(taba.at[idxv.at[pl.ds(128, 128)]],
                         rows.at[pl.ds(128, 128)], s2)
        pltpu.async_copy(taba.at[idxv.at[pl.ds(256, 32)]],
                         rows.at[pl.ds(256, 32)], s3)

    def consume(b, rows, s1, s2, s3):
        pltpu.make_async_copy(taba.at[pl.ds(0, 128)],
                             rows.at[pl.ds(0, 128)], s1).wait()
        pltpu.make_async_copy(taba.at[pl.ds(0, 128)],
                             rows.at[pl.ds(128, 128)], s2).wait()
        pltpu.make_async_copy(taba.at[pl.ds(0, 32)],
                             rows.at[pl.ds(256, 32)], s3).wait()

        def nrow(r, _2):
            rr = r * 9
            for q in range(5):
                acc = rows[rr, pl.ds(16 * q, 16)]
                for t in range(1, 9):
                    acc = acc + rows[rr + t, pl.ds(16 * q, 16)]
                hout[r, pl.ds(16 * q, 16)] = acc
            return 0

        lax.fori_loop(0, 32, nrow, 0)
        node0 = wid * 1568 + b * 32
        pltpu.sync_copy(hout.at[:, pl.ds(0, 32)], hws.at[0, pl.ds(node0, 32)])
        pltpu.sync_copy(hout.at[:, pl.ds(32, 32)], hws.at[1, pl.ds(node0, 32)])
        pltpu.sync_copy(hout.at[:, pl.ds(64, 16)], asad.at[pl.ds(node0, 32)])

    fire(0, xvA, idxA, rowsA, sa1, sa2, sa3)

    def it(j, _):
        fire(2 * j + 1, xvB, idxB, rowsB, sb1, sb2, sb3)
        consume(2 * j, rowsA, sa1, sa2, sa3)
        fire(2 * j + 2, xvA, idxA, rowsA, sa1, sa2, sa3)
        consume(2 * j + 1, rowsB, sb1, sb2, sb3)
        return 0

    lax.fori_loop(0, 24, it, 0)
    consume(48, rowsA, sa1, sa2, sa3)


def _edge_body(hwsf, asf, adf, mv, srcP, dstP, numP, denP,
               accS, denS, z32, zden, mvv,
               srcA, dstA, sixA, divA, gixA, dnxA, asvA, advA, exvA, rowsA,
               srcB, dstB, sixB, divB, gixB, dnxB, asvB, advB, exvB, rowsB,
               sA1, sA2, sA3, sB1, sB2, sB3, siA1, siA2, siB1, siB2,
               sdA, sdB):
    c = lax.axis_index("c")
    s = lax.axis_index("s")
    zf = jnp.zeros((16,), _F32)

    def z1(i, _):
        z32[i, pl.ds(0, 16)] = zf
        z32[i, pl.ds(16, 16)] = zf
        return 0

    lax.fori_loop(0, _EB, z1, 0)

    def z2(i, _):
        zden[pl.ds(i * 16, 16)] = zf
        return 0

    lax.fori_loop(0, _STRIPE // 16, z2, 0)
    r0 = s * _STRIPE
    for k in range(_STRIPE // _EB):
        pltpu.sync_copy(z32, accS.at[pl.ds(r0 + k * _EB, _EB)])
    pltpu.sync_copy(zden, denS.at[pl.ds(r0, _STRIPE)])
    pltpu.sync_copy(mv, mvv)
    plsc.subcore_barrier()

    coff = c * _PH0
    e0 = s * _NE_T
    nb1 = _NBLK - 1

    def idx_fire(b, sv, dv, s1, s2):
        base = e0 + b * _EB
        pltpu.async_copy(srcP.at[pl.ds(base, _EB)], sv, s1)
        pltpu.async_copy(dstP.at[pl.ds(base, _EB)], dv, s2)

    def idx_wait(sv, dv, s1, s2):
        pltpu.make_async_copy(srcP.at[pl.ds(0, _EB)], sv, s1).wait()
        pltpu.make_async_copy(dstP.at[pl.ds(0, _EB)], dv, s2).wait()

    def gfire(sv, dv, six, div, gix, asv, adv, rows, s1, s2, s3):
        for q in range(8):
            sq = sv[pl.ds(16 * q, 16)]
            six[pl.ds(16 * q, 16)] = sq
            gix[pl.ds(16 * q, 16)] = sq + coff
            div[pl.ds(16 * q, 16)] = dv[pl.ds(16 * q, 16)]
        pltpu.async_copy(asf.at[six], asv, s1)
        pltpu.async_copy(adf.at[div], adv, s2)
        pltpu.async_copy(hwsf.at[gix], rows, s3)

    def gwait(asv, adv, rows, s1, s2, s3):
        pltpu.make_async_copy(asf.at[pl.ds(0, _EB)], asv, s1).wait()
        pltpu.make_async_copy(adf.at[pl.ds(0, _EB)], adv, s2).wait()
        pltpu.make_async_copy(hwsf.at[pl.ds(0, _EB)], rows, s3).wait()

    def consume(div, dnx, asv, adv, exv, rows, sd):
        pltpu.make_async_copy(asf.at[pl.ds(0, _EB)], exv, sd).wait()
        mvec = mvv[pl.ds(0, 16)]
        for q in range(8):
            t = asv[pl.ds(16 * q, 16)] + adv[pl.ds(16 * q, 16)]
            e = jnp.where(t >= 0.0, t, 0.2 * t)
            exv[pl.ds(16 * q, 16)] = jnp.exp(e - mvec)

        def scg(g, _2):
            exvec = exv[pl.ds(g * 16, 16)]
            for r in range(16):
                sp = lax.gather(
                    exvec, jnp.full((16, 1), r, _I32),
                    lax.GatherDimensionNumbers(
                        offset_dims=(), collapsed_slice_dims=(0,),
                        start_index_map=(0,)),
                    slice_sizes=(1,),
                    mode=lax.GatherScatterMode.PROMISE_IN_BOUNDS)
                rr = g * 16 + r
                rows[rr, pl.ds(0, 16)] = rows[rr, pl.ds(0, 16)] * sp
                rows[rr, pl.ds(16, 16)] = rows[rr, pl.ds(16, 16)] * sp
            return 0

        lax.fori_loop(0, _EB // 16, scg, 0)
        pltpu.sync_copy(rows, accS.at[div], add=True)
        for q in range(8):
            dnx[pl.ds(16 * q, 16)] = div[pl.ds(16 * q, 16)]
        pltpu.async_copy(exv, denS.at[dnx], sd, add=True)

    pltpu.sync_copy(srcP.at[pl.ds(e0, _EB)], srcA)
    pltpu.sync_copy(dstP.at[pl.ds(e0, _EB)], dstA)
    gfire(srcA, dstA, sixA, divA, gixA, asvA, advA, rowsA, sA1, sA2, sA3)
    idx_fire(1, srcB, dstB, siB1, siB2)
    pltpu.async_copy(exvA, denS.at[pl.ds(_N + 432, _EB)], sdA)
    pltpu.async_copy(exvB, denS.at[pl.ds(_N + 432, _EB)], sdB)

    def it(j, _):
        g0 = 2 * j
        idx_wait(srcB, dstB, siB1, siB2)
        gfire(srcB, dstB, sixB, divB, gixB, asvB, advB, rowsB, sB1, sB2, sB3)
        idx_fire(jnp.minimum(g0 + 2, nb1), srcA, dstA, siA1, siA2)
        gwait(asvA, advA, rowsA, sA1, sA2, sA3)
        consume(divA, dnxA, asvA, advA, exvA, rowsA, sdA)
        idx_wait(srcA, dstA, siA1, siA2)
        gfire(srcA, dstA, sixA, divA, gixA, asvA, advA, rowsA, sA1, sA2, sA3)
        idx_fire(jnp.minimum(g0 + 3, nb1), srcB, dstB, siB1, siB2)
        gwait(asvB, advB, rowsB, sB1, sB2, sB3)
        consume(divB, dnxB, asvB, advB, exvB, rowsB, sdB)
        return 0

    lax.fori_loop(0, _NBLK // 2, it, 0)
    gwait(asvA, advA, rowsA, sA1, sA2, sA3)
    idx_wait(srcB, dstB, siB1, siB2)
    pltpu.make_async_copy(asf.at[pl.ds(0, _EB)], exvA, sdA).wait()
    pltpu.make_async_copy(asf.at[pl.ds(0, _EB)], exvB, sdB).wait()
    plsc.subcore_barrier()
    pltpu.sync_copy(accS.at[pl.ds(r0, _STRIPE)], numP.at[c, pl.ds(r0, _STRIPE)])
    pltpu.sync_copy(denS.at[pl.ds(r0, _STRIPE)], denP.at[c, pl.ds(r0, _STRIPE)])


def _pool_body(h3, bnd, psum, pmax, bndv, rbuf, rowb, osum, omax, sem):
    c = lax.axis_index("c")
    s = lax.axis_index("s")
    wid = c * 16 + s
    pltpu.sync_copy(bnd, bndv)
    iota = lax.iota(_I32, 16)

    def bval(t):
        blkoff = (t // 16) * 16
        vec = bndv[pl.ds(blkoff, 16)]
        return jnp.sum(jnp.where(iota == (t - blkoff), vec, 0))

    zf = jnp.zeros((16,), _F32)
    neg = jnp.full((16,), _NEG, _F32)
    for j in range(4):
        g = wid * 4 + j
        st = bval(g)
        en = bval(g + 1)
        cnt = en - st
        nfull = cnt // 16

        def fb(i, car):
            pltpu.sync_copy(h3.at[pl.ds(st + i * 16, 16)], rbuf)
            ss = list(car[:4])
            mm = list(car[4:])
            for r in range(16):
                for q in range(4):
                    v = rbuf[r, pl.ds(16 * q, 16)]
                    ss[q] = ss[q] + v
                    mm[q] = jnp.maximum(mm[q], v)
            return (*ss, *mm)

        car = lax.fori_loop(0, nfull, fb, (zf, zf, zf, zf, neg, neg, neg, neg))

        def tb(i, car2):
            pltpu.sync_copy(h3.at[pl.ds(st + nfull * 16 + i, 1)], rowb)
            ss = list(car2[:4])
            mm = list(car2[4:])
            for q in range(4):
                v = rowb[0, pl.ds(16 * q, 16)]
                ss[q] = ss[q] + v
                mm[q] = jnp.maximum(mm[q], v)
            return (*ss, *mm)

        car = lax.fori_loop(0, cnt - nfull * 16, tb, car)
        for q in range(4):
            osum[0, pl.ds(16 * q, 16)] = car[q]
            omax[0, pl.ds(16 * q, 16)] = car[4 + q]
        pltpu.sync_copy(osum, psum.at[pl.ds(g, 1)])
        pltpu.sync_copy(omax, pmax.at[pl.ds(g, 1)])


def _prep_tail(i, hout, wn_ref, ans_ref, adn_ref,
               hwn_ref, asn_ref, adn_out, mxs_ref, mxd_ref):
    hwn = jnp.dot(hout, wn_ref[...], preferred_element_type=_F32)
    hwn_ref[...] = jnp.stack([hwn[:, :32], hwn[:, 32:]], axis=0)
    a_s = jnp.sum(hwn * ans_ref[...], axis=1)
    a_d = jnp.sum(hwn * adn_ref[...], axis=1)
    asn_ref[...] = a_s.reshape(1, 1, _BN)
    adn_out[...] = a_d.reshape(1, 1, _BN)
    ms_prev = jnp.where(i == 0, _NEG, mxs_ref[...][0, 0])
    md_prev = jnp.where(i == 0, _NEG, mxd_ref[...][0, 0])
    mxs_ref[...] = jnp.maximum(ms_prev, jnp.max(a_s)).reshape(1, 1)
    mxd_ref[...] = jnp.maximum(md_prev, jnp.max(a_d)).reshape(1, 1)


def _tab_body(tab_ref, w_ref, a1s_ref, a1d_ref, o_ref):
    t = jnp.dot(tab_ref[...], w_ref[...], preferred_element_type=_F32)
    wa = jnp.sum(t * a1s_ref[...], axis=1, keepdims=True)
    wd = jnp.sum(t * a1d_ref[...], axis=1, keepdims=True)
    o_ref[...] = jnp.concatenate(
        [t, wa, wd, jnp.zeros((900, 14), _F32)], axis=1)


def _stats_body(asad_ref, batch_ref, asn_ref, adn_out, mxs_ref, mxd_ref,
                cnt_ref):
    i = pl.program_id(0)
    a_s = asad_ref[...][:, 0]
    a_d = asad_ref[...][:, 1]
    asn_ref[...] = a_s.reshape(1, 1, _BN)
    adn_out[...] = a_d.reshape(1, 1, _BN)
    ms_prev = jnp.where(i == 0, _NEG, mxs_ref[...][0, 0])
    md_prev = jnp.where(i == 0, _NEG, mxd_ref[...][0, 0])
    mxs_ref[...] = jnp.maximum(ms_prev, jnp.max(a_s)).reshape(1, 1)
    mxd_ref[...] = jnp.maximum(md_prev, jnp.max(a_d)).reshape(1, 1)
    b = batch_ref[...].reshape(_BN)
    oh = (b[:, None] == lax.broadcasted_iota(_I32, (_BN, _G), 1)).astype(_F32)
    c_prev = jnp.where(i == 0, jnp.zeros((1, _G), _F32), cnt_ref[...])
    cnt_ref[...] = c_prev + jnp.sum(oh, axis=0, keepdims=True)


def _combine(num0_ref, num1_ref, den_ref, hws0_ref, hws1_ref, as_ref, ad_ref,
             mxs_ref, mxd_ref, b_ref):
    mx = mxs_ref[...][0, 0] + mxd_ref[...][0, 0]
    m = jnp.maximum(mx, 0.2 * mx)
    t = as_ref[...].reshape(_BN) + ad_ref[...].reshape(_BN)
    e = jnp.where(t >= 0.0, t, 0.2 * t)
    exs = jnp.exp(e - m)
    hw = jnp.concatenate([hws0_ref[0], hws1_ref[0]], axis=1)
    num = jnp.concatenate([num0_ref[0], num1_ref[0]], axis=1)
    numv = num + exs[:, None] * hw
    denv = den_ref[...].reshape(_BN) + exs + 1e-16
    return jnp.maximum(numv / denv[:, None] + b_ref[...], 0.0)


def _cp_body(num0_ref, num1_ref, den_ref, hws0_ref, hws1_ref, as_ref, ad_ref,
             mxs_ref, mxd_ref, b_ref, wn_ref, ans_ref, adn_ref,
             hwn_ref, asn_ref, adn_out, mxs_out, mxd_out):
    i = pl.program_id(0)
    hout = _combine(num0_ref, num1_ref, den_ref, hws0_ref, hws1_ref, as_ref,
                    ad_ref, mxs_ref, mxd_ref, b_ref)
    _prep_tail(i, hout, wn_ref, ans_ref, adn_ref,
               hwn_ref, asn_ref, adn_out, mxs_out, mxd_out)


def _comb3_body(num0_ref, num1_ref, den_ref, hws0_ref, hws1_ref, as_ref,
                ad_ref, mxs_ref, mxd_ref, b_ref, cnt_ref, h3_ref, bnd_ref):
    h3_ref[...] = _combine(num0_ref, num1_ref, den_ref, hws0_ref, hws1_ref,
                           as_ref, ad_ref, mxs_ref, mxd_ref, b_ref)
    cnts = cnt_ref[...]
    ii = lax.broadcasted_iota(_I32, (_G, _G), 0)
    jj = lax.broadcasted_iota(_I32, (_G, _G), 1)
    tri = (ii <= jj).astype(_F32)
    cs = jnp.dot(cnts, tri, preferred_element_type=_F32)
    excl = (cs - cnts).astype(_I32)
    bnd_ref[...] = jnp.concatenate(
        [excl, jnp.full((1, 16), _N, _I32)], axis=1)


def _head_body(psum_ref, pmax_ref, cnt_ref, wl_ref, bl_ref, o_ref):
    cc = cnt_ref[...].reshape(_G, 1)
    ps = psum_ref[...]
    mean = ps / jnp.maximum(cc, 1.0)
    mx = jnp.where(cc > 0, pmax_ref[...], 0.0)
    pooled = jnp.concatenate([mean, ps, mx], axis=1)
    o_ref[...] = jnp.dot(pooled, wl_ref[...],
                         preferred_element_type=_F32) + bl_ref[...]


def _sc_mesh():
    return plsc.VectorSubcoreMesh(core_axis_name="c", subcore_axis_name="s")


def _emb_call(xf, taba, offs):
    kfn = pl.kernel(
        _emb_body,
        out_type=[jax.ShapeDtypeStruct((2, _PH0, 32), _F32),
                  jax.ShapeDtypeStruct((_PH0, 16), _F32)],
        mesh=_sc_mesh(),
        scratch_types=[
            pltpu.VMEM((288,), _I32),
            pltpu.VMEM((288,), _I32),
            pltpu.VMEM((288,), _I32),
            pltpu.VMEM((288, 80), _F32),
            pltpu.VMEM((288,), _I32),
            pltpu.VMEM((288,), _I32),
            pltpu.VMEM((288, 80), _F32),
            pltpu.VMEM((32, 80), _F32),
            pltpu.SemaphoreType.DMA,
            pltpu.SemaphoreType.DMA,
            pltpu.SemaphoreType.DMA,
            pltpu.SemaphoreType.DMA,
            pltpu.SemaphoreType.DMA,
            pltpu.SemaphoreType.DMA,
        ],
        compiler_params=pltpu.CompilerParams(use_tc_tiling_on_sc=False),
    )
    return kfn(xf, taba, offs)


def _tab_call(tabf, w1, a1s, a1d):
    return pl.pallas_call(
        _tab_body,
        out_shape=jax.ShapeDtypeStruct((900, 80), _F32),
    )(tabf, w1, a1s, a1d)


def _edge_call(hws, as3, ad3, mxs, mxd, srcP, dstP):
    asf = as3.reshape(_N)
    adf = ad3.reshape(_N)
    hwsf = hws.reshape(2 * _PH0, 32)
    mx = mxs[0, 0] + mxd[0, 0]
    m = jnp.maximum(mx, 0.2 * mx)
    mv = jnp.full((16,), 1.0, _F32) * m
    kfn = pl.kernel(
        _edge_body,
        out_type=[jax.ShapeDtypeStruct((2, _PN, 32), _F32),
                  jax.ShapeDtypeStruct((2, _PN), _F32)],
        mesh=_sc_mesh(),
        scratch_types=(
            [pltpu.VMEM_SHARED((_PN, 32), _F32),
             pltpu.VMEM_SHARED((_PN,), _F32),
             pltpu.VMEM((_EB, 32), _F32),
             pltpu.VMEM((_STRIPE,), _F32),
             pltpu.VMEM((16,), _F32)]
            + 2 * ([pltpu.VMEM((_EB,), _I32)] * 6
                   + [pltpu.VMEM((_EB,), _F32)] * 3
                   + [pltpu.VMEM((_EB, 32), _F32)])
            + [pltpu.SemaphoreType.DMA] * 12
        ),
        compiler_params=pltpu.CompilerParams(use_tc_tiling_on_sc=False),
    )
    return kfn(hwsf, asf, adf, mv, srcP, dstP)


def _pool_call(h3, bnd):
    kfn = pl.kernel(
        _pool_body,
        out_type=[jax.ShapeDtypeStruct((_G, 64), _F32),
                  jax.ShapeDtypeStruct((_G, 64), _F32)],
        mesh=_sc_mesh(),
        scratch_types=[
            pltpu.VMEM((144,), _I32),
            pltpu.VMEM((16, 64), _F32),
            pltpu.VMEM((1, 64), _F32),
            pltpu.VMEM((1, 64), _F32),
            pltpu.VMEM((1, 64), _F32),
            pltpu.SemaphoreType.DMA,
        ],
        compiler_params=pltpu.CompilerParams(use_tc_tiling_on_sc=False,
                                             needs_layout_passes=False),
    )
    return kfn(h3, bnd)


def _full(shape):
    return pl.BlockSpec(shape, lambda b: tuple(0 for _ in shape))


def _stats_call(asadE, batch3):
    return pl.pallas_call(
        _stats_body,
        grid=(_GRID,),
        in_specs=[
            pl.BlockSpec((_BN, 16), lambda b: (b, 0)),
            pl.BlockSpec((1, 1, _BN), lambda b: (b, 0, 0)),
        ],
        out_specs=[
            pl.BlockSpec((1, 1, _BN), lambda b: (b, 0, 0)),
            pl.BlockSpec((1, 1, _BN), lambda b: (b, 0, 0)),
            _full((1, 1)),
            _full((1, 1)),
            _full((1, _G)),
        ],
        out_shape=[
            jax.ShapeDtypeStruct((_GRID, 1, _BN), _F32),
            jax.ShapeDtypeStruct((_GRID, 1, _BN), _F32),
            jax.ShapeDtypeStruct((1, 1), _F32),
            jax.ShapeDtypeStruct((1, 1), _F32),
            jax.ShapeDtypeStruct((1, _G), _F32),
        ],
    )(asadE, batch3)


_CP_IN_SPECS = [
    pl.BlockSpec((1, _BN, 32), lambda b: (0, b, 0)),
    pl.BlockSpec((1, _BN, 32), lambda b: (1, b, 0)),
    pl.BlockSpec((1, 1, _BN), lambda b: (b, 0, 0)),
    pl.BlockSpec((1, _BN, 32), lambda b: (0, b, 0)),
    pl.BlockSpec((1, _BN, 32), lambda b: (1, b, 0)),
    pl.BlockSpec((1, 1, _BN), lambda b: (b, 0, 0)),
    pl.BlockSpec((1, 1, _BN), lambda b: (b, 0, 0)),
    _full((1, 1)),
    _full((1, 1)),
    _full((1, 64)),
]


def _cp_call(numP, denP, hws, as3, ad3, mxs, mxd, bb, wn, ans, adn):
    den3 = denP[0, :_N].reshape(_GRID, 1, _BN)
    return pl.pallas_call(
        _cp_body,
        grid=(_GRID,),
        in_specs=_CP_IN_SPECS + [
            _full((64, 64)),
            _full((1, 64)),
            _full((1, 64)),
        ],
        out_specs=[
            pl.BlockSpec((2, _BN, 32), lambda b: (0, b, 0)),
            pl.BlockSpec((1, 1, _BN), lambda b: (b, 0, 0)),
            pl.BlockSpec((1, 1, _BN), lambda b: (b, 0, 0)),
            _full((1, 1)),
            _full((1, 1)),
        ],
        out_shape=[
            jax.ShapeDtypeStruct((2, _PH0, 32), _F32),
            jax.ShapeDtypeStruct((_GRID, 1, _BN), _F32),
            jax.ShapeDtypeStruct((_GRID, 1, _BN), _F32),
            jax.ShapeDtypeStruct((1, 1), _F32),
            jax.ShapeDtypeStruct((1, 1), _F32),
        ],
    )(numP, numP, den3, hws, hws, as3, ad3, mxs, mxd, bb.reshape(1, 64),
      wn, ans.reshape(1, 64), adn.reshape(1, 64))


def _comb3_call(numP, denP, hws, as3, ad3, mxs, mxd, b3, counts):
    den3 = denP[0, :_N].reshape(_GRID, 1, _BN)
    return pl.pallas_call(
        _comb3_body,
        grid=(_GRID,),
        in_specs=_CP_IN_SPECS + [_full((1, _G))],
        out_specs=[
            pl.BlockSpec((_BN, 64), lambda b: (b, 0)),
            _full((1, 144)),
        ],
        out_shape=[
            jax.ShapeDtypeStruct((_N, 64), _F32),
            jax.ShapeDtypeStruct((1, 144), _I32),
        ],
    )(numP, numP, den3, hws, hws, as3, ad3, mxs, mxd, b3.reshape(1, 64),
      counts)


def _head_call(psum, pmax, counts, w_lin, b_lin):
    return pl.pallas_call(
        _head_body,
        out_shape=jax.ShapeDtypeStruct((_G, w_lin.shape[1]), _F32),
    )(psum, pmax, counts, w_lin, b_lin.reshape(1, -1))


def kernel(x, edge_index, batch, atom_emb, W1, a1_src, a1_dst, b1, W2, a2_src,
           a2_dst, b2, W3, a3_src, a3_dst, b3, W_lin, b_lin):
    x = x.astype(_I32)
    xf = jnp.pad(x.reshape(-1), (0, _PH0 * 9 - _N * 9))
    tabf = atom_emb.reshape(900, 128)
    offs = jnp.tile(jnp.arange(9, dtype=_I32) * 100, 32)
    taba = _tab_call(tabf, W1, a1_src.reshape(1, 64), a1_dst.reshape(1, 64))
    hws, asadE = _emb_call(xf, taba, offs)

    srcP = jnp.concatenate(
        [edge_index[0].astype(_I32), jnp.zeros((_EPAD - _E,), _I32)])
    dstP = jnp.concatenate(
        [edge_index[1].astype(_I32), jnp.full((_EPAD - _E,), _N, _I32)])
    batch3 = batch.astype(_I32).reshape(_GRID, 1, _BN)

    as3, ad3, mxs, mxd, counts = _stats_call(asadE, batch3)

    for (bb, wn, ans, adn) in ((b1, W2, a2_src, a2_dst),
                               (b2, W3, a3_src, a3_dst)):
        numP, denP = _edge_call(hws, as3, ad3, mxs, mxd, srcP, dstP)
        hws, as3, ad3, mxs, mxd = _cp_call(
            numP, denP, hws, as3, ad3, mxs, mxd, bb, wn, ans, adn)

    numP, denP = _edge_call(hws, as3, ad3, mxs, mxd, srcP, dstP)
    h3, bnd = _comb3_call(numP, denP, hws, as3, ad3, mxs, mxd, b3, counts)

    psum, pmax = _pool_call(h3, bnd.reshape(144))
    return _head_call(psum, pmax, counts, W_lin, b_lin)
